# Initial kernel scaffold; baseline (speedup 1.0000x reference)
import jax
import jax.numpy as jnp
from jax.experimental import pallas as pl


def kernel(x, edge_index_r0, edge_index_r1, edge_index_r2, W0, al0, ar0, b0, W1, al1, ar1, b1, W2, al2, ar2, b2):
    raise NotImplementedError("write your pallas kernel here")


# speedup vs baseline: 1.6960x; 1.6960x over previous
import functools

import jax
import jax.numpy as jnp
from jax import lax
from jax.experimental import pallas as pl
from jax.experimental.pallas import tpu as pltpu

N = 100000
E = 500000
D = 128
NBLK = 400


def _proj_body(x_ref, w_ref, a_ref, h_ref, s_ref):
    x = x_ref[...]
    h = jnp.dot(x, w_ref[...], preferred_element_type=jnp.float32)
    h_ref[...] = h
    s_ref[...] = jnp.dot(h, a_ref[...], preferred_element_type=jnp.float32)


def _project(x, Wcat, Acat):
    grid = N // NBLK
    return pl.pallas_call(
        _proj_body,
        grid=(grid,),
        in_specs=[
            pl.BlockSpec((NBLK, D), lambda i: (i, 0)),
            pl.BlockSpec((D, 3 * D), lambda i: (0, 0)),
            pl.BlockSpec((3 * D, 8), lambda i: (0, 0)),
        ],
        out_specs=[
            pl.BlockSpec((NBLK, 3 * D), lambda i: (i, 0)),
            pl.BlockSpec((NBLK, 8), lambda i: (i, 0)),
        ],
        out_shape=[
            jax.ShapeDtypeStruct((N, 3 * D), jnp.float32),
            jax.ShapeDtypeStruct((N, 8), jnp.float32),
        ],
    )(x, Wcat, Acat)


def _combine_body(u0_ref, u1_ref, u2_ref, b_ref, o_ref):
    o_ref[...] = (u0_ref[...] + u1_ref[...] + u2_ref[...]) / 3.0 + b_ref[...]


def _combine(u0, u1, u2, bmean):
    grid = N // NBLK
    return pl.pallas_call(
        _combine_body,
        grid=(grid,),
        in_specs=[
            pl.BlockSpec((NBLK, D), lambda i: (i, 0)),
            pl.BlockSpec((NBLK, D), lambda i: (i, 0)),
            pl.BlockSpec((NBLK, D), lambda i: (i, 0)),
            pl.BlockSpec((1, D), lambda i: (0, 0)),
        ],
        out_specs=pl.BlockSpec((NBLK, D), lambda i: (i, 0)),
        out_shape=jax.ShapeDtypeStruct((N, D), jnp.float32),
    )(u0, u1, u2, bmean)


def _edge_phase(h, el, er, src, dst):
    e = jax.nn.leaky_relu(el[src] + er[dst], negative_slope=0.2)
    ee = jnp.exp(e)
    denom = jax.ops.segment_sum(ee, dst, num_segments=N)
    u = jax.ops.segment_sum(ee[:, None] * h[src], dst, num_segments=N)
    return u / denom[:, None]


def kernel(x, edge_index_r0, edge_index_r1, edge_index_r2,
           W0, al0, ar0, b0, W1, al1, ar1, b1, W2, al2, ar2, b2):
    Wcat = jnp.concatenate([W0, W1, W2], axis=1)
    Acat = jnp.zeros((3 * D, 8), jnp.float32)
    for r, (al, ar) in enumerate(((al0, ar0), (al1, ar1), (al2, ar2))):
        Acat = Acat.at[r * D:(r + 1) * D, 2 * r].set(al)
        Acat = Acat.at[r * D:(r + 1) * D, 2 * r + 1].set(ar)
    h, scores = _project(x, Wcat, Acat)

    us = []
    for r, ei in enumerate((edge_index_r0, edge_index_r1, edge_index_r2)):
        src, dst = ei[0], ei[1]
        us.append(_edge_phase(h[:, r * D:(r + 1) * D],
                              scores[:, 2 * r], scores[:, 2 * r + 1], src, dst))
    bsum = (b0 + b1 + b2).reshape(1, D) / 3.0
    return _combine(us[0], us[1], us[2], bsum)


# speedup vs baseline: 3.9341x; 2.3197x over previous
import functools

import jax
import jax.numpy as jnp
from jax import lax
from jax.experimental import pallas as pl
from jax.experimental.pallas import tpu as pltpu
from jax.experimental.pallas import tpu_sc as plsc

N = 100000
E = 500000
D = 128
NR = 3
NBLK = 400
NCH = (E + 127) // 128
EPAD = NCH * 128
NP = 100096
NSL = NP // 16
NC_ROWS = 8192
NCHUNK = (N + NC_ROWS - 1) // NC_ROWS
UPAD = NCHUNK * NC_ROWS
ROWS_PT = NC_ROWS // 16
KA1 = (NCH + 31) // 32
KB = (NCH + 15) // 16
FLUSH_AT = 112

_MESH = plsc.VectorSubcoreMesh(core_axis_name="c", subcore_axis_name="s",
                               num_cores=2, num_subcores=16)
_i32 = jnp.int32


def _proj_body(x_ref, w_ref, a_ref, h_ref, s_ref):
    x = x_ref[...]
    h = jnp.dot(x, w_ref[0], preferred_element_type=jnp.float32)
    h_ref[0] = h
    s_ref[0] = jnp.dot(h, a_ref[0], preferred_element_type=jnp.float32)


def _project(x, Wstack, Astack):
    return pl.pallas_call(
        _proj_body,
        grid=(N // NBLK, NR),
        in_specs=[
            pl.BlockSpec((NBLK, D), lambda i, r: (i, 0)),
            pl.BlockSpec((1, D, D), lambda i, r: (r, 0, 0)),
            pl.BlockSpec((1, D, 2), lambda i, r: (r, 0, 0)),
        ],
        out_specs=[
            pl.BlockSpec((1, NBLK, D), lambda i, r: (r, i, 0)),
            pl.BlockSpec((1, NBLK, 2), lambda i, r: (r, i, 0)),
        ],
        out_shape=[
            jax.ShapeDtypeStruct((NR, N, D), jnp.float32),
            jax.ShapeDtypeStruct((NR, N, 2), jnp.float32),
        ],
    )(x, Wstack, Astack)


def _combine_body(u_ref, b_ref, o_ref):
    o_ref[...] = u_ref[...] * (1.0 / 3.0) + b_ref[...]


def _combine(u, bsum):
    return pl.pallas_call(
        _combine_body,
        grid=(N // NBLK,),
        in_specs=[
            pl.BlockSpec((NBLK, D), lambda i: (i, 0)),
            pl.BlockSpec((1, D), lambda i: (0, 0)),
        ],
        out_specs=pl.BlockSpec((NBLK, D), lambda i: (i, 0)),
        out_shape=jax.ShapeDtypeStruct((N, D), jnp.float32),
    )(u, bsum)


def _a1_body(src_hbm, dst_hbm, el_hbm, er_hbm, ee_hbm, denp_hbm,
             el_sh, er_sh, den_sh, buf, src_v, dst_v, eev, sem):
    c = lax.axis_index("c")
    s = lax.axis_index("s")
    w32 = c * 16 + s
    zero16 = jnp.zeros((16,), jnp.float32)

    base = s * NSL
    for r in range(NR):
        pltpu.sync_copy(el_hbm.at[pl.ds(r * NP + base, NSL)], buf)
        pltpu.sync_copy(buf, el_sh[r].at[pl.ds(base, NSL)])
        pltpu.sync_copy(er_hbm.at[pl.ds(r * NP + base, NSL)], buf)
        pltpu.sync_copy(buf, er_sh[r].at[pl.ds(base, NSL)])
    def _zero_buf(i):
        buf[pl.ds(i * 16, 16)] = zero16
    pl.loop(0, NSL // 16)(_zero_buf)
    for r in range(NR):
        pltpu.sync_copy(buf, den_sh[r].at[pl.ds(base, NSL)])
    plsc.subcore_barrier()

    def _chunk(r, k):
        j = w32 + 32 * k

        @pl.when(j < NCH)
        def _():
            eb = r * EPAD + j * 128
            pltpu.sync_copy(src_hbm.at[pl.ds(eb, 128)], src_v)
            pltpu.sync_copy(dst_hbm.at[pl.ds(eb, 128)], dst_v)
            pltpu.async_copy(el_sh[r].at[src_v], eev, sem).wait()
            pltpu.async_copy(er_sh[r].at[dst_v], buf.at[pl.ds(0, 128)], sem).wait()

            def _grp(g):
                v = eev[pl.ds(g * 16, 16)] + buf[pl.ds(g * 16, 16)]
                v = jnp.where(v > 0, v, v * jnp.float32(0.2))
                eev[pl.ds(g * 16, 16)] = jnp.exp(v)
            pl.loop(0, 8)(_grp)

            pltpu.sync_copy(eev, ee_hbm.at[pl.ds(eb, 128)])
            pltpu.sync_copy(eev, den_sh[r].at[dst_v], add=True)

    for r in range(NR):
        pl.loop(0, KA1)(functools.partial(_chunk, r))

    plsc.subcore_barrier()
    for r in range(NR):
        pltpu.sync_copy(den_sh[r].at[pl.ds(base, NSL)], buf)
        pltpu.sync_copy(buf, denp_hbm.at[pl.ds((c * NR + r) * NP + base, NSL)])


def _phase_a1(src3, dst3, el3, er3):
    f = pl.kernel(
        _a1_body,
        out_type=[
            jax.ShapeDtypeStruct((NR * EPAD,), jnp.float32),
            jax.ShapeDtypeStruct((2 * NR * NP,), jnp.float32),
        ],
        mesh=_MESH,
        scratch_types=[
            [pltpu.VMEM_SHARED((NP,), jnp.float32) for _ in range(NR)],
            [pltpu.VMEM_SHARED((NP,), jnp.float32) for _ in range(NR)],
            [pltpu.VMEM_SHARED((NP,), jnp.float32) for _ in range(NR)],
            pltpu.VMEM((NSL,), jnp.float32),
            pltpu.VMEM((128,), _i32),
            pltpu.VMEM((128,), _i32),
            pltpu.VMEM((128,), jnp.float32),
            pltpu.SemaphoreType.DMA,
        ],
        compiler_params=pltpu.CompilerParams(needs_layout_passes=False),
    )
    return f(src3, dst3, el3, er3)


def _a2_body(dst_hbm, ee_hbm, denp_hbm, alpha_hbm,
             den_sh, v0, v1, dst_v, eev, dv, sem):
    c = lax.axis_index("c")
    s = lax.axis_index("s")
    w32 = c * 16 + s
    base = s * NSL

    for r in range(NR):
        pltpu.sync_copy(denp_hbm.at[pl.ds(r * NP + base, NSL)], v0)
        pltpu.sync_copy(denp_hbm.at[pl.ds((NR + r) * NP + base, NSL)], v1)

        def _sum(i):
            v0[pl.ds(i * 16, 16)] = v0[pl.ds(i * 16, 16)] + v1[pl.ds(i * 16, 16)]
        pl.loop(0, NSL // 16)(_sum)
        pltpu.sync_copy(v0, den_sh[r].at[pl.ds(base, NSL)])
    plsc.subcore_barrier()

    def _chunk(r, k):
        j = w32 + 32 * k

        @pl.when(j < NCH)
        def _():
            eb = r * EPAD + j * 128
            pltpu.sync_copy(dst_hbm.at[pl.ds(eb, 128)], dst_v)
            pltpu.sync_copy(ee_hbm.at[pl.ds(eb, 128)], eev)
            pltpu.async_copy(den_sh[r].at[dst_v], dv, sem).wait()

            def _grp(g):
                eev[pl.ds(g * 16, 16)] = eev[pl.ds(g * 16, 16)] / dv[pl.ds(g * 16, 16)]
            pl.loop(0, 8)(_grp)
            pltpu.sync_copy(eev, alpha_hbm.at[pl.ds(eb, 128)])

    for r in range(NR):
        pl.loop(0, KA1)(functools.partial(_chunk, r))


def _phase_a2(dst3, ee3, denp):
    f = pl.kernel(
        _a2_body,
        out_type=jax.ShapeDtypeStruct((NR * EPAD,), jnp.float32),
        mesh=_MESH,
        scratch_types=[
            [pltpu.VMEM_SHARED((NP,), jnp.float32) for _ in range(NR)],
            pltpu.VMEM((NSL,), jnp.float32),
            pltpu.VMEM((NSL,), jnp.float32),
            pltpu.VMEM((128,), _i32),
            pltpu.VMEM((128,), jnp.float32),
            pltpu.VMEM((128,), jnp.float32),
            pltpu.SemaphoreType.DMA,
        ],
        compiler_params=pltpu.CompilerParams(needs_layout_passes=False),
    )
    return f(dst3, ee3, denp)


def _b_body(src_hbm, dst_hbm, alpha_hbm, h_hbm, u_hbm,
            chunk_sh, st_src, st_dst, st_a, rows, zbuf,
            src_v, dst_v, a_v, sem):
    c = lax.axis_index("c")
    s = lax.axis_index("s")
    zero16 = jnp.zeros((16,), jnp.float32)
    zero16i = jnp.zeros((16,), _i32)
    iota16 = lax.iota(_i32, 16)

    for q in range(8):
        st_src[pl.ds(q * 16, 16)] = zero16i
        st_dst[pl.ds(q * 16, 16)] = zero16i
        st_a[pl.ds(q * 16, 16)] = zero16

    def _zrow(i):
        for q in range(8):
            zbuf[i, pl.ds(q * 16, 16)] = zero16
    pl.loop(0, 64)(_zrow)

    def _flush():
        pltpu.async_copy(h_hbm.at[st_src], rows, sem).wait()

        def _scale(i):
            av = plsc.load_gather(st_a, [zero16i + i])
            for q in range(8):
                rows[i, pl.ds(q * 16, 16)] = rows[i, pl.ds(q * 16, 16)] * av
        pl.loop(0, 128)(_scale)
        pltpu.sync_copy(rows, chunk_sh.at[st_dst], add=True)
        for q in range(8):
            st_a[pl.ds(q * 16, 16)] = zero16

    def _per_chunk(k, carry):
        chunk = 2 * k + c

        @pl.when(chunk < NCHUNK)
        def _():
            for i in range(8):
                pltpu.sync_copy(zbuf, chunk_sh.at[pl.ds(s * ROWS_PT + i * 64, 64), :])
        plsc.subcore_barrier()

        lo = chunk * NC_ROWS
        hi = jnp.minimum(lo + NC_ROWS, N)

        def _scan_chunk(r, k2, off):
            j = s + 16 * k2
            eb = r * EPAD + j * 128
            pltpu.sync_copy(src_hbm.at[pl.ds(eb, 128)], src_v)
            pltpu.sync_copy(dst_hbm.at[pl.ds(eb, 128)], dst_v)
            pltpu.sync_copy(alpha_hbm.at[pl.ds(eb, 128)], a_v)

            def _grp(g, offv):
                vd = dst_v[pl.ds(g * 16, 16)]
                m = (vd >= lo) & (vd < hi)
                cs = jnp.cumsum(m.astype(_i32))
                cntv = lax.gather(
                    cs, (zero16i + 15)[:, None],
                    dimension_numbers=lax.GatherDimensionNumbers(
                        offset_dims=(), collapsed_slice_dims=(0,),
                        start_index_map=(0,)),
                    slice_sizes=(1,),
                    mode=lax.GatherScatterMode.PROMISE_IN_BOUNDS)
                pos = offv + cs - 1
                plsc.store_scatter(st_src, [pos],
                                   src_v[pl.ds(g * 16, 16)] + jnp.int32(r * N),
                                   mask=m)
                plsc.store_scatter(st_dst, [pos], vd - lo, mask=m)
                plsc.store_scatter(st_a, [pos], a_v[pl.ds(g * 16, 16)], mask=m)
                offv = offv + cntv
                flushp = jnp.any(offv >= FLUSH_AT)
                pl.when(flushp)(_flush)
                return jnp.where(flushp, 0, offv)

            return lax.fori_loop(0, 8, _grp, off)

        kb_s = (NCH - s + 15) // 16

        @pl.when(chunk < NCHUNK)
        def _():
            o = jnp.zeros((16,), _i32)
            for r in range(NR):
                o = lax.fori_loop(0, kb_s, functools.partial(_scan_chunk, r), o)
            _flush()
        plsc.subcore_barrier()

        @pl.when(chunk < NCHUNK)
        def _():
            rb = s * ROWS_PT
            for i in range(ROWS_PT // 128):
                pltpu.sync_copy(chunk_sh.at[pl.ds(rb + i * 128, 128), :], rows)
                pltpu.sync_copy(
                    rows, u_hbm.at[pl.ds(chunk * NC_ROWS + rb + i * 128, 128), :])
        plsc.subcore_barrier()
        return carry

    lax.fori_loop(0, (NCHUNK + 1) // 2, _per_chunk, 0)


def _phase_b(src3, dst3, alpha3, h3f):
    f = pl.kernel(
        _b_body,
        out_type=jax.ShapeDtypeStruct((UPAD, D), jnp.float32),
        mesh=_MESH,
        scratch_types=[
            pltpu.VMEM_SHARED((NC_ROWS, D), jnp.float32),
            pltpu.VMEM((128,), _i32),
            pltpu.VMEM((128,), _i32),
            pltpu.VMEM((128,), jnp.float32),
            pltpu.VMEM((128, D), jnp.float32),
            pltpu.VMEM((64, D), jnp.float32),
            pltpu.VMEM((128,), _i32),
            pltpu.VMEM((128,), _i32),
            pltpu.VMEM((128,), jnp.float32),
            pltpu.SemaphoreType.DMA,
        ],
        compiler_params=pltpu.CompilerParams(needs_layout_passes=False),
    )
    return f(src3, dst3, alpha3, h3f)


def kernel(x, edge_index_r0, edge_index_r1, edge_index_r2,
           W0, al0, ar0, b0, W1, al1, ar1, b1, W2, al2, ar2, b2):
    Wstack = jnp.stack([W0, W1, W2])
    Astack = jnp.stack([jnp.stack([al0, ar0], axis=1),
                        jnp.stack([al1, ar1], axis=1),
                        jnp.stack([al2, ar2], axis=1)])
    h3, scores = _project(x, Wstack, Astack)

    ei = jnp.stack([edge_index_r0, edge_index_r1, edge_index_r2])
    src3 = jnp.pad(ei[:, 0, :], ((0, 0), (0, EPAD - E))).reshape(-1)
    dst3 = jnp.pad(ei[:, 1, :], ((0, 0), (0, EPAD - E)),
                   constant_values=N).reshape(-1)
    el3 = jnp.pad(scores[:, :, 0], ((0, 0), (0, NP - N))).reshape(-1)
    er3 = jnp.pad(scores[:, :, 1], ((0, 0), (0, NP - N))).reshape(-1)

    ee3, denp = _phase_a1(src3, dst3, el3, er3)
    alpha3 = _phase_a2(dst3, ee3, denp)
    u = _phase_b(src3, dst3, alpha3, h3.reshape(NR * N, D))

    bsum = ((b0 + b1 + b2) / 3.0).reshape(1, D)
    return _combine(u[:N], bsum)


# speedup vs baseline: 5.8974x; 1.4991x over previous
import functools

import jax
import jax.numpy as jnp
from jax import lax
from jax.experimental import pallas as pl
from jax.experimental.pallas import tpu as pltpu
from jax.experimental.pallas import tpu_sc as plsc

N = 100000
E = 500000
D = 128
NR = 3
NBLK = 400
EPAD = 500736
NCH = EPAD // 128
NCHB = 3 * EPAD // 1024
NP = 100096
NSL = NP // 16
NC_ROWS = 12544
NCHUNK = 8
UPAD = NCHUNK * NC_ROWS
ROWS_PT = NC_ROWS // 16
KA1 = (NCH + 31) // 32
FLUSH_AT = 112

_MESH = plsc.VectorSubcoreMesh(core_axis_name="c", subcore_axis_name="s",
                               num_cores=2, num_subcores=16)
_i32 = jnp.int32


def _proj_body(x_ref, w_ref, a_ref, h_ref, s_ref):
    x = x_ref[...]
    h = jnp.dot(x, w_ref[0], preferred_element_type=jnp.float32)
    h_ref[0] = h
    s_ref[0] = jnp.dot(h, a_ref[0], preferred_element_type=jnp.float32)


def _project(x, Wstack, Astack):
    return pl.pallas_call(
        _proj_body,
        grid=(N // NBLK, NR),
        in_specs=[
            pl.BlockSpec((NBLK, D), lambda i, r: (i, 0)),
            pl.BlockSpec((1, D, D), lambda i, r: (r, 0, 0)),
            pl.BlockSpec((1, D, 2), lambda i, r: (r, 0, 0)),
        ],
        out_specs=[
            pl.BlockSpec((1, NBLK, D), lambda i, r: (r, i, 0)),
            pl.BlockSpec((1, NBLK, 2), lambda i, r: (r, i, 0)),
        ],
        out_shape=[
            jax.ShapeDtypeStruct((NR, N, D), jnp.float32),
            jax.ShapeDtypeStruct((NR, N, 2), jnp.float32),
        ],
    )(x, Wstack, Astack)


def _combine_body(u_ref, b_ref, o_ref):
    o_ref[...] = u_ref[...] * (1.0 / 3.0) + b_ref[...]


def _combine(u, bsum):
    return pl.pallas_call(
        _combine_body,
        grid=(N // NBLK,),
        in_specs=[
            pl.BlockSpec((NBLK, D), lambda i: (i, 0)),
            pl.BlockSpec((1, D), lambda i: (0, 0)),
        ],
        out_specs=pl.BlockSpec((NBLK, D), lambda i: (i, 0)),
        out_shape=jax.ShapeDtypeStruct((N, D), jnp.float32),
    )(u, bsum)


def _a1_body(src_hbm, dst_hbm, el_hbm, er_hbm, ee_hbm, denp_hbm,
             el_sh, er_sh, den_sh, buf, src_v, dst_v, eev, sem):
    c = lax.axis_index("c")
    s = lax.axis_index("s")
    w32 = c * 16 + s
    zero16 = jnp.zeros((16,), jnp.float32)

    base = s * NSL
    for r in range(NR):
        pltpu.sync_copy(el_hbm.at[pl.ds(r * NP + base, NSL)], buf)
        pltpu.sync_copy(buf, el_sh[r].at[pl.ds(base, NSL)])
        pltpu.sync_copy(er_hbm.at[pl.ds(r * NP + base, NSL)], buf)
        pltpu.sync_copy(buf, er_sh[r].at[pl.ds(base, NSL)])
    def _zero_buf(i):
        buf[pl.ds(i * 16, 16)] = zero16
    pl.loop(0, NSL // 16)(_zero_buf)
    for r in range(NR):
        pltpu.sync_copy(buf, den_sh[r].at[pl.ds(base, NSL)])
    plsc.subcore_barrier()

    def _chunk(r, k):
        j = w32 + 32 * k

        @pl.when(j < NCH)
        def _():
            eb = r * EPAD + j * 128
            pltpu.sync_copy(src_hbm.at[pl.ds(eb, 128)], src_v)
            pltpu.sync_copy(dst_hbm.at[pl.ds(eb, 128)], dst_v)
            pltpu.async_copy(el_sh[r].at[src_v], eev, sem).wait()
            pltpu.async_copy(er_sh[r].at[dst_v], buf.at[pl.ds(0, 128)], sem).wait()

            def _grp(g):
                v = eev[pl.ds(g * 16, 16)] + buf[pl.ds(g * 16, 16)]
                v = jnp.where(v > 0, v, v * jnp.float32(0.2))
                eev[pl.ds(g * 16, 16)] = jnp.exp(v)
            pl.loop(0, 8)(_grp)

            pltpu.sync_copy(eev, ee_hbm.at[pl.ds(eb, 128)])
            pltpu.sync_copy(eev, den_sh[r].at[dst_v], add=True)

    for r in range(NR):
        pl.loop(0, KA1)(functools.partial(_chunk, r))

    plsc.subcore_barrier()
    for r in range(NR):
        pltpu.sync_copy(den_sh[r].at[pl.ds(base, NSL)], buf)
        pltpu.sync_copy(buf, denp_hbm.at[pl.ds((c * NR + r) * NP + base, NSL)])


def _phase_a1(src3, dst3, el3, er3):
    f = pl.kernel(
        _a1_body,
        out_type=[
            jax.ShapeDtypeStruct((NR * EPAD,), jnp.float32),
            jax.ShapeDtypeStruct((2 * NR * NP,), jnp.float32),
        ],
        mesh=_MESH,
        scratch_types=[
            [pltpu.VMEM_SHARED((NP,), jnp.float32) for _ in range(NR)],
            [pltpu.VMEM_SHARED((NP,), jnp.float32) for _ in range(NR)],
            [pltpu.VMEM_SHARED((NP,), jnp.float32) for _ in range(NR)],
            pltpu.VMEM((NSL,), jnp.float32),
            pltpu.VMEM((128,), _i32),
            pltpu.VMEM((128,), _i32),
            pltpu.VMEM((128,), jnp.float32),
            pltpu.SemaphoreType.DMA,
        ],
        compiler_params=pltpu.CompilerParams(needs_layout_passes=False),
    )
    return f(src3, dst3, el3, er3)


def _a2_body(src_hbm, dst_hbm, ee_hbm, denp_hbm, packed_hbm,
             den_sh, v0, v1, src_v, dst_v, eev, dv, av, sem):
    c = lax.axis_index("c")
    s = lax.axis_index("s")
    w32 = c * 16 + s
    base = s * NSL

    for r in range(NR):
        pltpu.sync_copy(denp_hbm.at[pl.ds(r * NP + base, NSL)], v0)
        pltpu.sync_copy(denp_hbm.at[pl.ds((NR + r) * NP + base, NSL)], v1)

        def _sum(i):
            v0[pl.ds(i * 16, 16)] = v0[pl.ds(i * 16, 16)] + v1[pl.ds(i * 16, 16)]
        pl.loop(0, NSL // 16)(_sum)
        pltpu.sync_copy(v0, den_sh[r].at[pl.ds(base, NSL)])
    plsc.subcore_barrier()

    def _chunk(r, k):
        j = w32 + 32 * k

        @pl.when(j < NCH)
        def _():
            eb = r * EPAD + j * 128
            pb = (eb // 1024) * 3072 + (eb % 1024)
            pltpu.sync_copy(src_hbm.at[pl.ds(eb, 128)], src_v)
            pltpu.sync_copy(dst_hbm.at[pl.ds(eb, 128)], dst_v)
            pltpu.sync_copy(ee_hbm.at[pl.ds(eb, 128)], eev)
            pltpu.async_copy(den_sh[r].at[dst_v], dv, sem).wait()

            def _grp(g):
                sl = pl.ds(g * 16, 16)
                src_v[sl] = src_v[sl] + jnp.int32(r * N)
                av[sl] = plsc.bitcast(eev[sl] / dv[sl], _i32)
            pl.loop(0, 8)(_grp)
            pltpu.sync_copy(src_v, packed_hbm.at[pl.ds(pb, 128)])
            pltpu.sync_copy(dst_v, packed_hbm.at[pl.ds(pb + 1024, 128)])
            pltpu.sync_copy(av, packed_hbm.at[pl.ds(pb + 2048, 128)])

    for r in range(NR):
        pl.loop(0, KA1)(functools.partial(_chunk, r))


def _phase_a2(src3, dst3, ee3, denp):
    f = pl.kernel(
        _a2_body,
        out_type=jax.ShapeDtypeStruct((NCHB * 3072,), _i32),
        mesh=_MESH,
        scratch_types=[
            [pltpu.VMEM_SHARED((NP,), jnp.float32) for _ in range(NR)],
            pltpu.VMEM((NSL,), jnp.float32),
            pltpu.VMEM((NSL,), jnp.float32),
            pltpu.VMEM((128,), _i32),
            pltpu.VMEM((128,), _i32),
            pltpu.VMEM((128,), jnp.float32),
            pltpu.VMEM((128,), jnp.float32),
            pltpu.VMEM((128,), _i32),
            pltpu.SemaphoreType.DMA,
        ],
        compiler_params=pltpu.CompilerParams(needs_layout_passes=False),
    )
    return f(src3, dst3, ee3, denp)


def _b_body(packed_hbm, h_hbm, u_hbm,
            chunk_sh, st_src, st_dst, st_a, rows, zbuf, pbuf, sem):
    c = lax.axis_index("c")
    s = lax.axis_index("s")
    zero16 = jnp.zeros((16,), jnp.float32)
    zero16i = jnp.zeros((16,), _i32)

    for q in range(8):
        st_src[pl.ds(q * 16, 16)] = zero16i
        st_dst[pl.ds(q * 16, 16)] = zero16i
        st_a[pl.ds(q * 16, 16)] = zero16

    def _zrow(i):
        for q in range(8):
            zbuf[i, pl.ds(q * 16, 16)] = zero16
    pl.loop(0, 64)(_zrow)

    def _flush():
        pltpu.async_copy(h_hbm.at[st_src], rows, sem).wait()

        def _scale(i):
            av = plsc.load_gather(st_a, [zero16i + i])
            for q in range(8):
                rows[i, pl.ds(q * 16, 16)] = rows[i, pl.ds(q * 16, 16)] * av
        pl.loop(0, 128)(_scale)
        pltpu.sync_copy(rows, chunk_sh.at[st_dst], add=True)
        for q in range(8):
            st_a[pl.ds(q * 16, 16)] = zero16

    kbt = (NCHB - s + 15) // 16

    def _per_chunk(k, carry):
        chunk = 2 * k + c
        for i in range(12):
            pltpu.sync_copy(zbuf, chunk_sh.at[pl.ds(s * ROWS_PT + i * 64, 64), :])
        pltpu.sync_copy(zbuf.at[pl.ds(0, 16), :],
                        chunk_sh.at[pl.ds(s * ROWS_PT + 768, 16), :])
        plsc.subcore_barrier()

        lo = chunk * NC_ROWS
        hi = jnp.minimum(lo + NC_ROWS, N)

        def _scan_chunk(k2, off):
            t = s + 16 * k2
            pltpu.sync_copy(packed_hbm.at[pl.ds(t * 3072, 3072)], pbuf)

            def _grp(g, offv):
                vd = pbuf[pl.ds(1024 + g * 16, 16)]
                m = (vd >= lo) & (vd < hi)
                cs = jnp.cumsum(m.astype(_i32))
                cntv = lax.gather(
                    cs, (zero16i + 15)[:, None],
                    dimension_numbers=lax.GatherDimensionNumbers(
                        offset_dims=(), collapsed_slice_dims=(0,),
                        start_index_map=(0,)),
                    slice_sizes=(1,),
                    mode=lax.GatherScatterMode.PROMISE_IN_BOUNDS)
                pos = offv + cs - 1
                plsc.store_scatter(st_src, [pos], pbuf[pl.ds(g * 16, 16)], mask=m)
                plsc.store_scatter(st_dst, [pos], vd - lo, mask=m)
                plsc.store_scatter(st_a, [pos],
                                   plsc.bitcast(pbuf[pl.ds(2048 + g * 16, 16)],
                                                jnp.float32), mask=m)
                offv = offv + cntv
                flushp = jnp.any(offv >= FLUSH_AT)
                pl.when(flushp)(_flush)
                return jnp.where(flushp, 0, offv)

            return lax.fori_loop(0, 64, _grp, off)

        o = lax.fori_loop(0, kbt, _scan_chunk, jnp.zeros((16,), _i32))
        _flush()
        plsc.subcore_barrier()

        rb = s * ROWS_PT
        for i in range(6):
            pltpu.sync_copy(chunk_sh.at[pl.ds(rb + i * 128, 128), :], rows)
            pltpu.sync_copy(
                rows, u_hbm.at[pl.ds(chunk * NC_ROWS + rb + i * 128, 128), :])
        pltpu.sync_copy(chunk_sh.at[pl.ds(rb + 768, 16), :], rows.at[pl.ds(0, 16), :])
        pltpu.sync_copy(rows.at[pl.ds(0, 16), :],
                        u_hbm.at[pl.ds(chunk * NC_ROWS + rb + 768, 16), :])
        plsc.subcore_barrier()
        return carry

    lax.fori_loop(0, NCHUNK // 2, _per_chunk, 0)


def _phase_b(packed, h3f):
    f = pl.kernel(
        _b_body,
        out_type=jax.ShapeDtypeStruct((UPAD, D), jnp.float32),
        mesh=_MESH,
        scratch_types=[
            pltpu.VMEM_SHARED((NC_ROWS, D), jnp.float32),
            pltpu.VMEM((128,), _i32),
            pltpu.VMEM((128,), _i32),
            pltpu.VMEM((128,), jnp.float32),
            pltpu.VMEM((128, D), jnp.float32),
            pltpu.VMEM((64, D), jnp.float32),
            pltpu.VMEM((3072,), _i32),
            pltpu.SemaphoreType.DMA,
        ],
        compiler_params=pltpu.CompilerParams(needs_layout_passes=False),
    )
    return f(packed, h3f)


def kernel(x, edge_index_r0, edge_index_r1, edge_index_r2,
           W0, al0, ar0, b0, W1, al1, ar1, b1, W2, al2, ar2, b2):
    Wstack = jnp.stack([W0, W1, W2])
    Astack = jnp.stack([jnp.stack([al0, ar0], axis=1),
                        jnp.stack([al1, ar1], axis=1),
                        jnp.stack([al2, ar2], axis=1)])
    h3, scores = _project(x, Wstack, Astack)

    ei = jnp.stack([edge_index_r0, edge_index_r1, edge_index_r2])
    src3 = jnp.pad(ei[:, 0, :], ((0, 0), (0, EPAD - E))).reshape(-1)
    dst3 = jnp.pad(ei[:, 1, :], ((0, 0), (0, EPAD - E)),
                   constant_values=N).reshape(-1)
    el3 = jnp.pad(scores[:, :, 0], ((0, 0), (0, NP - N))).reshape(-1)
    er3 = jnp.pad(scores[:, :, 1], ((0, 0), (0, NP - N))).reshape(-1)

    ee3, denp = _phase_a1(src3, dst3, el3, er3)
    packed = _phase_a2(src3, dst3, ee3, denp)
    u = _phase_b(packed, h3.reshape(NR * N, D))

    bsum = ((b0 + b1 + b2) / 3.0).reshape(1, D)
    return _combine(u[:N], bsum)


# speedup vs baseline: 8.4371x; 1.4306x over previous
import functools

import jax
import jax.numpy as jnp
from jax import lax
from jax.experimental import pallas as pl
from jax.experimental.pallas import tpu as pltpu
from jax.experimental.pallas import tpu_sc as plsc

N = 100000
E = 500000
D = 128
NR = 3
NBLK = 400
EPAD = 500736
NCH = EPAD // 128
NCHB = 3 * EPAD // 1024
NP = 100096
NSL = NP // 16
NC_ROWS = 11264
NCHUNK = 9
UPAD = NCHUNK * NC_ROWS
ROWS_PT = NC_ROWS // 16
KA1 = (NCH + 31) // 32
FLUSH_AT = 240

_MESH = plsc.VectorSubcoreMesh(core_axis_name="c", subcore_axis_name="s",
                               num_cores=2, num_subcores=16)
_i32 = jnp.int32


def _proj_body(x_ref, w_ref, a_ref, h_ref, s_ref):
    x = x_ref[...]
    h = jnp.dot(x, w_ref[0], preferred_element_type=jnp.float32)
    h_ref[0] = h
    s_ref[0] = jnp.dot(h, a_ref[0], preferred_element_type=jnp.float32)


def _project(x, Wstack, Astack):
    return pl.pallas_call(
        _proj_body,
        grid=(N // NBLK, NR),
        in_specs=[
            pl.BlockSpec((NBLK, D), lambda i, r: (i, 0)),
            pl.BlockSpec((1, D, D), lambda i, r: (r, 0, 0)),
            pl.BlockSpec((1, D, 2), lambda i, r: (r, 0, 0)),
        ],
        out_specs=[
            pl.BlockSpec((1, NBLK, D), lambda i, r: (r, i, 0)),
            pl.BlockSpec((1, NBLK, 2), lambda i, r: (r, i, 0)),
        ],
        out_shape=[
            jax.ShapeDtypeStruct((NR, N, D), jnp.float32),
            jax.ShapeDtypeStruct((NR, N, 2), jnp.float32),
        ],
    )(x, Wstack, Astack)


def _combine_body(u_ref, b_ref, o_ref):
    o_ref[...] = u_ref[...] * (1.0 / 3.0) + b_ref[...]


def _combine(u, bsum):
    return pl.pallas_call(
        _combine_body,
        grid=(N // NBLK,),
        in_specs=[
            pl.BlockSpec((NBLK, D), lambda i: (i, 0)),
            pl.BlockSpec((1, D), lambda i: (0, 0)),
        ],
        out_specs=pl.BlockSpec((NBLK, D), lambda i: (i, 0)),
        out_shape=jax.ShapeDtypeStruct((N, D), jnp.float32),
    )(u, bsum)


def _a1_body(src_hbm, dst_hbm, el_hbm, er_hbm, ee_hbm, denp_hbm,
             el_sh, er_sh, den_sh, buf, src_v, dst_v, eev, sem):
    c = lax.axis_index("c")
    s = lax.axis_index("s")
    w32 = c * 16 + s
    zero16 = jnp.zeros((16,), jnp.float32)

    base = s * NSL
    for r in range(NR):
        pltpu.sync_copy(el_hbm.at[pl.ds(r * NP + base, NSL)], buf)
        pltpu.sync_copy(buf, el_sh[r].at[pl.ds(base, NSL)])
        pltpu.sync_copy(er_hbm.at[pl.ds(r * NP + base, NSL)], buf)
        pltpu.sync_copy(buf, er_sh[r].at[pl.ds(base, NSL)])
    def _zero_buf(i):
        buf[pl.ds(i * 16, 16)] = zero16
    pl.loop(0, NSL // 16)(_zero_buf)
    for r in range(NR):
        pltpu.sync_copy(buf, den_sh[r].at[pl.ds(base, NSL)])
    plsc.subcore_barrier()

    def _chunk(r, k):
        j = w32 + 32 * k

        @pl.when(j < NCH)
        def _():
            eb = r * EPAD + j * 128
            pltpu.sync_copy(src_hbm.at[pl.ds(eb, 128)], src_v)
            pltpu.sync_copy(dst_hbm.at[pl.ds(eb, 128)], dst_v)
            pltpu.async_copy(el_sh[r].at[src_v], eev, sem).wait()
            pltpu.async_copy(er_sh[r].at[dst_v], buf.at[pl.ds(0, 128)], sem).wait()

            def _grp(g):
                v = eev[pl.ds(g * 16, 16)] + buf[pl.ds(g * 16, 16)]
                v = jnp.where(v > 0, v, v * jnp.float32(0.2))
                eev[pl.ds(g * 16, 16)] = jnp.exp(v)
            pl.loop(0, 8)(_grp)

            pltpu.sync_copy(eev, ee_hbm.at[pl.ds(eb, 128)])
            pltpu.sync_copy(eev, den_sh[r].at[dst_v], add=True)

    for r in range(NR):
        pl.loop(0, KA1)(functools.partial(_chunk, r))

    plsc.subcore_barrier()
    for r in range(NR):
        pltpu.sync_copy(den_sh[r].at[pl.ds(base, NSL)], buf)
        pltpu.sync_copy(buf, denp_hbm.at[pl.ds((c * NR + r) * NP + base, NSL)])


def _phase_a1(src3, dst3, el3, er3):
    f = pl.kernel(
        _a1_body,
        out_type=[
            jax.ShapeDtypeStruct((NR * EPAD,), jnp.float32),
            jax.ShapeDtypeStruct((2 * NR * NP,), jnp.float32),
        ],
        mesh=_MESH,
        scratch_types=[
            [pltpu.VMEM_SHARED((NP,), jnp.float32) for _ in range(NR)],
            [pltpu.VMEM_SHARED((NP,), jnp.float32) for _ in range(NR)],
            [pltpu.VMEM_SHARED((NP,), jnp.float32) for _ in range(NR)],
            pltpu.VMEM((NSL,), jnp.float32),
            pltpu.VMEM((128,), _i32),
            pltpu.VMEM((128,), _i32),
            pltpu.VMEM((128,), jnp.float32),
            pltpu.SemaphoreType.DMA,
        ],
        compiler_params=pltpu.CompilerParams(needs_layout_passes=False),
    )
    return f(src3, dst3, el3, er3)


def _a2_body(src_hbm, dst_hbm, ee_hbm, denp_hbm, packed_hbm,
             den_sh, v0, v1, src_v, dst_v, eev, dv, av, sem):
    c = lax.axis_index("c")
    s = lax.axis_index("s")
    w32 = c * 16 + s
    base = s * NSL

    for r in range(NR):
        pltpu.sync_copy(denp_hbm.at[pl.ds(r * NP + base, NSL)], v0)
        pltpu.sync_copy(denp_hbm.at[pl.ds((NR + r) * NP + base, NSL)], v1)

        def _sum(i):
            v0[pl.ds(i * 16, 16)] = v0[pl.ds(i * 16, 16)] + v1[pl.ds(i * 16, 16)]
        pl.loop(0, NSL // 16)(_sum)
        pltpu.sync_copy(v0, den_sh[r].at[pl.ds(base, NSL)])
    plsc.subcore_barrier()

    def _chunk(r, k):
        j = w32 + 32 * k

        @pl.when(j < NCH)
        def _():
            eb = r * EPAD + j * 128
            pb = (eb // 1024) * 3072 + (eb % 1024)
            pltpu.sync_copy(src_hbm.at[pl.ds(eb, 128)], src_v)
            pltpu.sync_copy(dst_hbm.at[pl.ds(eb, 128)], dst_v)
            pltpu.sync_copy(ee_hbm.at[pl.ds(eb, 128)], eev)
            pltpu.async_copy(den_sh[r].at[dst_v], dv, sem).wait()

            def _grp(g):
                sl = pl.ds(g * 16, 16)
                src_v[sl] = src_v[sl] + jnp.int32(r * N)
                av[sl] = plsc.bitcast(eev[sl] / dv[sl], _i32)
            pl.loop(0, 8)(_grp)
            pltpu.sync_copy(src_v, packed_hbm.at[pl.ds(pb, 128)])
            pltpu.sync_copy(dst_v, packed_hbm.at[pl.ds(pb + 1024, 128)])
            pltpu.sync_copy(av, packed_hbm.at[pl.ds(pb + 2048, 128)])

    for r in range(NR):
        pl.loop(0, KA1)(functools.partial(_chunk, r))


def _phase_a2(src3, dst3, ee3, denp):
    f = pl.kernel(
        _a2_body,
        out_type=jax.ShapeDtypeStruct((NCHB * 3072,), _i32),
        mesh=_MESH,
        scratch_types=[
            [pltpu.VMEM_SHARED((NP,), jnp.float32) for _ in range(NR)],
            pltpu.VMEM((NSL,), jnp.float32),
            pltpu.VMEM((NSL,), jnp.float32),
            pltpu.VMEM((128,), _i32),
            pltpu.VMEM((128,), _i32),
            pltpu.VMEM((128,), jnp.float32),
            pltpu.VMEM((128,), jnp.float32),
            pltpu.VMEM((128,), _i32),
            pltpu.SemaphoreType.DMA,
        ],
        compiler_params=pltpu.CompilerParams(needs_layout_passes=False),
    )
    return f(src3, dst3, ee3, denp)


def _b_body(packed_hbm, h_hbm, u_hbm,
            chunk_sh, st_src0, st_src1, st_dst0, st_dst1, st_a,
            rows0, rows1, pbuf, sem, sem2):
    c = lax.axis_index("c")
    s = lax.axis_index("s")
    zero16 = jnp.zeros((16,), jnp.float32)
    zero16i = jnp.zeros((16,), _i32)

    for q in range(8):
        st_src0[pl.ds(q * 16, 16)] = zero16i
        st_src1[pl.ds(q * 16, 16)] = zero16i
        st_dst0[pl.ds(q * 16, 16)] = zero16i
        st_dst1[pl.ds(q * 16, 16)] = zero16i
    for q in range(16):
        st_a[pl.ds(q * 16, 16)] = zero16

    def _flush():
        d0 = pltpu.async_copy(h_hbm.at[st_src0], rows0, sem)
        d1 = pltpu.async_copy(h_hbm.at[st_src1], rows1, sem2)
        d0.wait()
        d1.wait()

        def _scale0(i):
            av = plsc.load_gather(st_a, [zero16i + i])
            for q in range(8):
                rows0[i, pl.ds(q * 16, 16)] = rows0[i, pl.ds(q * 16, 16)] * av
        pl.loop(0, 128, unroll=2)(_scale0)

        def _scale1(i):
            av = plsc.load_gather(st_a, [zero16i + (i + 128)])
            for q in range(8):
                rows1[i, pl.ds(q * 16, 16)] = rows1[i, pl.ds(q * 16, 16)] * av
        pl.loop(0, 128, unroll=2)(_scale1)
        pltpu.sync_copy(rows0, chunk_sh.at[st_dst0], add=True)
        pltpu.sync_copy(rows1, chunk_sh.at[st_dst1], add=True)
        for q in range(16):
            st_a[pl.ds(q * 16, 16)] = zero16

    kbt = (NCHB - s + 15) // 16

    def _per_chunk(k, carry):
        chunk = 2 * k + c

        @pl.when(chunk < NCHUNK)
        def _():
            def _zrow(i):
                for q in range(8):
                    rows0[i, pl.ds(q * 16, 16)] = zero16
            pl.loop(0, 128)(_zrow)
            for i in range(5):
                pltpu.sync_copy(rows0,
                                chunk_sh.at[pl.ds(s * ROWS_PT + i * 128, 128), :])
            pltpu.sync_copy(rows0.at[pl.ds(0, 64), :],
                            chunk_sh.at[pl.ds(s * ROWS_PT + 640, 64), :])
        plsc.subcore_barrier()

        lo = chunk * NC_ROWS
        hi = jnp.minimum(lo + NC_ROWS, N)

        def _scan_chunk(k2, off):
            t = s + 16 * k2
            pltpu.sync_copy(packed_hbm.at[pl.ds(t * 3072, 3072)], pbuf)

            def _grp(g, offv):
                vd = pbuf[pl.ds(1024 + g * 16, 16)]
                m = (vd >= lo) & (vd < hi)
                cs = jnp.cumsum(m.astype(_i32))
                cntv = lax.gather(
                    cs, (zero16i + 15)[:, None],
                    dimension_numbers=lax.GatherDimensionNumbers(
                        offset_dims=(), collapsed_slice_dims=(0,),
                        start_index_map=(0,)),
                    slice_sizes=(1,),
                    mode=lax.GatherScatterMode.PROMISE_IN_BOUNDS)
                pos = offv + cs - 1
                m0 = m & (pos < 128)
                m1 = m & (pos >= 128)
                vs = pbuf[pl.ds(g * 16, 16)]
                plsc.store_scatter(st_src0, [pos], vs, mask=m0)
                plsc.store_scatter(st_src1, [pos - 128], vs, mask=m1)
                vdl = vd - lo
                plsc.store_scatter(st_dst0, [pos], vdl, mask=m0)
                plsc.store_scatter(st_dst1, [pos - 128], vdl, mask=m1)
                plsc.store_scatter(st_a, [pos],
                                   plsc.bitcast(pbuf[pl.ds(2048 + g * 16, 16)],
                                                jnp.float32), mask=m)
                offv = offv + cntv
                flushp = jnp.any(offv >= FLUSH_AT)
                pl.when(flushp)(_flush)
                return jnp.where(flushp, 0, offv)

            return lax.fori_loop(0, 64, _grp, off)

        @pl.when(chunk < NCHUNK)
        def _():
            lax.fori_loop(0, kbt, _scan_chunk, jnp.zeros((16,), _i32))
            _flush()
        plsc.subcore_barrier()

        @pl.when(chunk < NCHUNK)
        def _():
            rb = s * ROWS_PT
            for i in range(5):
                pltpu.sync_copy(chunk_sh.at[pl.ds(rb + i * 128, 128), :], rows0)
                pltpu.sync_copy(
                    rows0, u_hbm.at[pl.ds(chunk * NC_ROWS + rb + i * 128, 128), :])
            pltpu.sync_copy(chunk_sh.at[pl.ds(rb + 640, 64), :],
                            rows0.at[pl.ds(0, 64), :])
            pltpu.sync_copy(rows0.at[pl.ds(0, 64), :],
                            u_hbm.at[pl.ds(chunk * NC_ROWS + rb + 640, 64), :])
        plsc.subcore_barrier()
        return carry

    lax.fori_loop(0, (NCHUNK + 1) // 2, _per_chunk, 0)


def _phase_b(packed, h3f):
    f = pl.kernel(
        _b_body,
        out_type=jax.ShapeDtypeStruct((UPAD, D), jnp.float32),
        mesh=_MESH,
        scratch_types=[
            pltpu.VMEM_SHARED((NC_ROWS, D), jnp.float32),
            pltpu.VMEM((128,), _i32),
            pltpu.VMEM((128,), _i32),
            pltpu.VMEM((128,), _i32),
            pltpu.VMEM((128,), _i32),
            pltpu.VMEM((256,), jnp.float32),
            pltpu.VMEM((128, D), jnp.float32),
            pltpu.VMEM((128, D), jnp.float32),
            pltpu.VMEM((3072,), _i32),
            pltpu.SemaphoreType.DMA,
            pltpu.SemaphoreType.DMA,
        ],
        compiler_params=pltpu.CompilerParams(needs_layout_passes=False),
    )
    return f(packed, h3f)


def kernel(x, edge_index_r0, edge_index_r1, edge_index_r2,
           W0, al0, ar0, b0, W1, al1, ar1, b1, W2, al2, ar2, b2):
    Wstack = jnp.stack([W0, W1, W2])
    Astack = jnp.stack([jnp.stack([al0, ar0], axis=1),
                        jnp.stack([al1, ar1], axis=1),
                        jnp.stack([al2, ar2], axis=1)])
    h3, scores = _project(x, Wstack, Astack)

    ei = jnp.stack([edge_index_r0, edge_index_r1, edge_index_r2])
    src3 = jnp.pad(ei[:, 0, :], ((0, 0), (0, EPAD - E))).reshape(-1)
    dst3 = jnp.pad(ei[:, 1, :], ((0, 0), (0, EPAD - E)),
                   constant_values=N).reshape(-1)
    el3 = jnp.pad(scores[:, :, 0], ((0, 0), (0, NP - N))).reshape(-1)
    er3 = jnp.pad(scores[:, :, 1], ((0, 0), (0, NP - N))).reshape(-1)

    ee3, denp = _phase_a1(src3, dst3, el3, er3)
    packed = _phase_a2(src3, dst3, ee3, denp)
    u = _phase_b(packed, h3.reshape(NR * N, D))

    bsum = ((b0 + b1 + b2) / 3.0).reshape(1, D)
    return _combine(u[:N], bsum)


# speedup vs baseline: 8.4945x; 1.0068x over previous
import functools

import jax
import jax.numpy as jnp
from jax import lax
from jax.experimental import pallas as pl
from jax.experimental.pallas import tpu as pltpu
from jax.experimental.pallas import tpu_sc as plsc

N = 100000
E = 500000
D = 128
NR = 3
NBLK = 400
EPAD = 500736
NCH = EPAD // 128
NCHB = 3 * EPAD // 1024
NP = 100096
NSL = NP // 16
NC_ROWS = 11264
NCHUNK = 9
UPAD = NCHUNK * NC_ROWS
ROWS_PT = NC_ROWS // 16
KA1 = (NCH + 31) // 32
FLUSH_AT = 240

_MESH = plsc.VectorSubcoreMesh(core_axis_name="c", subcore_axis_name="s",
                               num_cores=2, num_subcores=16)
_i32 = jnp.int32


def _proj_body(x_ref, w_ref, a_ref, h_ref, s_ref):
    x = x_ref[...]
    h = jnp.dot(x, w_ref[0], preferred_element_type=jnp.float32)
    h_ref[0] = h
    s_ref[0] = jnp.dot(h, a_ref[0], preferred_element_type=jnp.float32)


def _project(x, Wstack, Astack):
    return pl.pallas_call(
        _proj_body,
        grid=(N // NBLK, NR),
        in_specs=[
            pl.BlockSpec((NBLK, D), lambda i, r: (i, 0)),
            pl.BlockSpec((1, D, D), lambda i, r: (r, 0, 0)),
            pl.BlockSpec((1, D, 2), lambda i, r: (r, 0, 0)),
        ],
        out_specs=[
            pl.BlockSpec((1, NBLK, D), lambda i, r: (r, i, 0)),
            pl.BlockSpec((1, NBLK, 2), lambda i, r: (r, i, 0)),
        ],
        out_shape=[
            jax.ShapeDtypeStruct((NR, N, D), jnp.float32),
            jax.ShapeDtypeStruct((NR, N, 2), jnp.float32),
        ],
    )(x, Wstack, Astack)


def _combine_body(u_ref, b_ref, o_ref):
    o_ref[...] = u_ref[...] * (1.0 / 3.0) + b_ref[...]


def _combine(u, bsum):
    return pl.pallas_call(
        _combine_body,
        grid=(N // NBLK,),
        in_specs=[
            pl.BlockSpec((NBLK, D), lambda i: (i, 0)),
            pl.BlockSpec((1, D), lambda i: (0, 0)),
        ],
        out_specs=pl.BlockSpec((NBLK, D), lambda i: (i, 0)),
        out_shape=jax.ShapeDtypeStruct((N, D), jnp.float32),
    )(u, bsum)


def _a1_body(src_hbm, dst_hbm, el_hbm, er_hbm, ee_hbm, denp_hbm,
             el_sh, er_sh, den_sh, buf, src_v, dst_v, eev, sem):
    c = lax.axis_index("c")
    s = lax.axis_index("s")
    w32 = c * 16 + s
    zero16 = jnp.zeros((16,), jnp.float32)

    base = s * NSL
    for r in range(NR):
        pltpu.sync_copy(el_hbm.at[pl.ds(r * NP + base, NSL)], buf)
        pltpu.sync_copy(buf, el_sh[r].at[pl.ds(base, NSL)])
        pltpu.sync_copy(er_hbm.at[pl.ds(r * NP + base, NSL)], buf)
        pltpu.sync_copy(buf, er_sh[r].at[pl.ds(base, NSL)])
    def _zero_buf(i):
        buf[pl.ds(i * 16, 16)] = zero16
    pl.loop(0, NSL // 16)(_zero_buf)
    for r in range(NR):
        pltpu.sync_copy(buf, den_sh[r].at[pl.ds(base, NSL)])
    plsc.subcore_barrier()

    def _chunk(r, k):
        j = w32 + 32 * k

        @pl.when(j < NCH)
        def _():
            eb = r * EPAD + j * 128
            pltpu.sync_copy(src_hbm.at[pl.ds(eb, 128)], src_v)
            pltpu.sync_copy(dst_hbm.at[pl.ds(eb, 128)], dst_v)
            pltpu.async_copy(el_sh[r].at[src_v], eev, sem).wait()
            pltpu.async_copy(er_sh[r].at[dst_v], buf.at[pl.ds(0, 128)], sem).wait()

            def _grp(g):
                v = eev[pl.ds(g * 16, 16)] + buf[pl.ds(g * 16, 16)]
                v = jnp.where(v > 0, v, v * jnp.float32(0.2))
                eev[pl.ds(g * 16, 16)] = jnp.exp(v)
            pl.loop(0, 8)(_grp)

            pltpu.sync_copy(eev, ee_hbm.at[pl.ds(eb, 128)])
            pltpu.sync_copy(eev, den_sh[r].at[dst_v], add=True)

    for r in range(NR):
        pl.loop(0, KA1)(functools.partial(_chunk, r))

    plsc.subcore_barrier()
    for r in range(NR):
        pltpu.sync_copy(den_sh[r].at[pl.ds(base, NSL)], buf)
        pltpu.sync_copy(buf, denp_hbm.at[pl.ds((c * NR + r) * NP + base, NSL)])


def _phase_a1(src3, dst3, el3, er3):
    f = pl.kernel(
        _a1_body,
        out_type=[
            jax.ShapeDtypeStruct((NR * EPAD,), jnp.float32),
            jax.ShapeDtypeStruct((2 * NR * NP,), jnp.float32),
        ],
        mesh=_MESH,
        scratch_types=[
            [pltpu.VMEM_SHARED((NP,), jnp.float32) for _ in range(NR)],
            [pltpu.VMEM_SHARED((NP,), jnp.float32) for _ in range(NR)],
            [pltpu.VMEM_SHARED((NP,), jnp.float32) for _ in range(NR)],
            pltpu.VMEM((NSL,), jnp.float32),
            pltpu.VMEM((128,), _i32),
            pltpu.VMEM((128,), _i32),
            pltpu.VMEM((128,), jnp.float32),
            pltpu.SemaphoreType.DMA,
        ],
        compiler_params=pltpu.CompilerParams(needs_layout_passes=False),
    )
    return f(src3, dst3, el3, er3)


def _a2_body(src_hbm, dst_hbm, ee_hbm, denp_hbm, packed_hbm,
             den_sh, v0, v1, src_v, dst_v, eev, dv, av, sem):
    c = lax.axis_index("c")
    s = lax.axis_index("s")
    w32 = c * 16 + s
    base = s * NSL

    for r in range(NR):
        pltpu.sync_copy(denp_hbm.at[pl.ds(r * NP + base, NSL)], v0)
        pltpu.sync_copy(denp_hbm.at[pl.ds((NR + r) * NP + base, NSL)], v1)

        def _sum(i):
            v0[pl.ds(i * 16, 16)] = v0[pl.ds(i * 16, 16)] + v1[pl.ds(i * 16, 16)]
        pl.loop(0, NSL // 16)(_sum)
        pltpu.sync_copy(v0, den_sh[r].at[pl.ds(base, NSL)])
    plsc.subcore_barrier()

    def _chunk(r, k):
        j = w32 + 32 * k

        @pl.when(j < NCH)
        def _():
            eb = r * EPAD + j * 128
            pb = (eb // 1024) * 3072 + (eb % 1024)
            pltpu.sync_copy(src_hbm.at[pl.ds(eb, 128)], src_v)
            pltpu.sync_copy(dst_hbm.at[pl.ds(eb, 128)], dst_v)
            pltpu.sync_copy(ee_hbm.at[pl.ds(eb, 128)], eev)
            pltpu.async_copy(den_sh[r].at[dst_v], dv, sem).wait()

            def _grp(g):
                sl = pl.ds(g * 16, 16)
                src_v[sl] = src_v[sl] + jnp.int32(r * N)
                av[sl] = plsc.bitcast(eev[sl] / dv[sl], _i32)
            pl.loop(0, 8)(_grp)
            pltpu.sync_copy(src_v, packed_hbm.at[pl.ds(pb, 128)])
            pltpu.sync_copy(dst_v, packed_hbm.at[pl.ds(pb + 1024, 128)])
            pltpu.sync_copy(av, packed_hbm.at[pl.ds(pb + 2048, 128)])

    for r in range(NR):
        pl.loop(0, KA1)(functools.partial(_chunk, r))


def _phase_a2(src3, dst3, ee3, denp):
    f = pl.kernel(
        _a2_body,
        out_type=jax.ShapeDtypeStruct((NCHB * 3072,), _i32),
        mesh=_MESH,
        scratch_types=[
            [pltpu.VMEM_SHARED((NP,), jnp.float32) for _ in range(NR)],
            pltpu.VMEM((NSL,), jnp.float32),
            pltpu.VMEM((NSL,), jnp.float32),
            pltpu.VMEM((128,), _i32),
            pltpu.VMEM((128,), _i32),
            pltpu.VMEM((128,), jnp.float32),
            pltpu.VMEM((128,), jnp.float32),
            pltpu.VMEM((128,), _i32),
            pltpu.SemaphoreType.DMA,
        ],
        compiler_params=pltpu.CompilerParams(needs_layout_passes=False),
    )
    return f(src3, dst3, ee3, denp)


def _b_body(packed_hbm, h_hbm, u_hbm,
            chunk_sh, st_src0, st_src1, st_dst0, st_dst1, st_a,
            rows0, rows1, pbuf, pbuf1, sem, sem2, semp0, semp1):
    c = lax.axis_index("c")
    s = lax.axis_index("s")
    zero16 = jnp.zeros((16,), jnp.float32)
    zero16i = jnp.zeros((16,), _i32)

    for q in range(8):
        st_src0[pl.ds(q * 16, 16)] = zero16i
        st_src1[pl.ds(q * 16, 16)] = zero16i
        st_dst0[pl.ds(q * 16, 16)] = zero16i
        st_dst1[pl.ds(q * 16, 16)] = zero16i
    for q in range(16):
        st_a[pl.ds(q * 16, 16)] = zero16

    def _flush():
        d0 = pltpu.async_copy(h_hbm.at[st_src0], rows0, sem)
        d1 = pltpu.async_copy(h_hbm.at[st_src1], rows1, sem2)
        d0.wait()
        d1.wait()

        def _scale0(i):
            av = plsc.load_gather(st_a, [zero16i + i])
            for q in range(8):
                rows0[i, pl.ds(q * 16, 16)] = rows0[i, pl.ds(q * 16, 16)] * av
        pl.loop(0, 128, unroll=2)(_scale0)

        def _scale1(i):
            av = plsc.load_gather(st_a, [zero16i + (i + 128)])
            for q in range(8):
                rows1[i, pl.ds(q * 16, 16)] = rows1[i, pl.ds(q * 16, 16)] * av
        pl.loop(0, 128, unroll=2)(_scale1)
        pltpu.sync_copy(rows0, chunk_sh.at[st_dst0], add=True)
        pltpu.sync_copy(rows1, chunk_sh.at[st_dst1], add=True)
        for q in range(16):
            st_a[pl.ds(q * 16, 16)] = zero16

    kbt = (NCHB - s + 15) // 16

    def _per_chunk(k, carry):
        chunk = 2 * k + c

        @pl.when(chunk < NCHUNK)
        def _():
            def _zrow(i):
                for q in range(8):
                    rows0[i, pl.ds(q * 16, 16)] = zero16
            pl.loop(0, 128)(_zrow)
            for i in range(5):
                pltpu.sync_copy(rows0,
                                chunk_sh.at[pl.ds(s * ROWS_PT + i * 128, 128), :])
            pltpu.sync_copy(rows0.at[pl.ds(0, 64), :],
                            chunk_sh.at[pl.ds(s * ROWS_PT + 640, 64), :])
        plsc.subcore_barrier()

        lo = chunk * NC_ROWS
        hi = jnp.minimum(lo + NC_ROWS, N)

        def _mk_grp(buf):
            def _grp(g, offv):
                vd = buf[pl.ds(1024 + g * 16, 16)]
                m = (vd >= lo) & (vd < hi)
                cs = jnp.cumsum(m.astype(_i32))
                cntv = lax.gather(
                    cs, (zero16i + 15)[:, None],
                    dimension_numbers=lax.GatherDimensionNumbers(
                        offset_dims=(), collapsed_slice_dims=(0,),
                        start_index_map=(0,)),
                    slice_sizes=(1,),
                    mode=lax.GatherScatterMode.PROMISE_IN_BOUNDS)
                pos = offv + cs - 1
                m0 = m & (pos < 128)
                m1 = m & (pos >= 128)
                vs = buf[pl.ds(g * 16, 16)]
                plsc.store_scatter(st_src0, [pos], vs, mask=m0)
                plsc.store_scatter(st_src1, [pos - 128], vs, mask=m1)
                vdl = vd - lo
                plsc.store_scatter(st_dst0, [pos], vdl, mask=m0)
                plsc.store_scatter(st_dst1, [pos - 128], vdl, mask=m1)
                plsc.store_scatter(st_a, [pos],
                                   plsc.bitcast(buf[pl.ds(2048 + g * 16, 16)],
                                                jnp.float32), mask=m)
                offv = offv + cntv
                flushp = jnp.any(offv >= FLUSH_AT)
                pl.when(flushp)(_flush)
                return jnp.where(flushp, 0, offv)
            return _grp

        def _pair(k3, off):
            t0 = (s + 32 * k3) * 3072
            d0 = pltpu.async_copy(packed_hbm.at[pl.ds(t0, 3072)], pbuf, semp0)
            d1 = pltpu.async_copy(packed_hbm.at[pl.ds(t0 + 16 * 3072, 3072)],
                                  pbuf1, semp1)
            d0.wait()
            off = lax.fori_loop(0, 64, _mk_grp(pbuf), off)
            d1.wait()
            return lax.fori_loop(0, 64, _mk_grp(pbuf1), off)

        def _single(k2, off):
            t = s + 16 * k2
            pltpu.sync_copy(packed_hbm.at[pl.ds(t * 3072, 3072)], pbuf)
            return lax.fori_loop(0, 64, _mk_grp(pbuf), off)

        @pl.when(chunk < NCHUNK)
        def _():
            o = lax.fori_loop(0, kbt // 2, _pair, jnp.zeros((16,), _i32))
            o = lax.fori_loop(2 * (kbt // 2), kbt, _single, o)
            _flush()
        plsc.subcore_barrier()

        @pl.when(chunk < NCHUNK)
        def _():
            rb = s * ROWS_PT
            for i in range(5):
                pltpu.sync_copy(chunk_sh.at[pl.ds(rb + i * 128, 128), :], rows0)
                pltpu.sync_copy(
                    rows0, u_hbm.at[pl.ds(chunk * NC_ROWS + rb + i * 128, 128), :])
            pltpu.sync_copy(chunk_sh.at[pl.ds(rb + 640, 64), :],
                            rows0.at[pl.ds(0, 64), :])
            pltpu.sync_copy(rows0.at[pl.ds(0, 64), :],
                            u_hbm.at[pl.ds(chunk * NC_ROWS + rb + 640, 64), :])
        plsc.subcore_barrier()
        return carry

    lax.fori_loop(0, (NCHUNK + 1) // 2, _per_chunk, 0)


def _phase_b(packed, h3f):
    f = pl.kernel(
        _b_body,
        out_type=jax.ShapeDtypeStruct((UPAD, D), jnp.float32),
        mesh=_MESH,
        scratch_types=[
            pltpu.VMEM_SHARED((NC_ROWS, D), jnp.float32),
            pltpu.VMEM((128,), _i32),
            pltpu.VMEM((128,), _i32),
            pltpu.VMEM((128,), _i32),
            pltpu.VMEM((128,), _i32),
            pltpu.VMEM((256,), jnp.float32),
            pltpu.VMEM((128, D), jnp.float32),
            pltpu.VMEM((128, D), jnp.float32),
            pltpu.VMEM((3072,), _i32),
            pltpu.VMEM((3072,), _i32),
            pltpu.SemaphoreType.DMA,
            pltpu.SemaphoreType.DMA,
            pltpu.SemaphoreType.DMA,
            pltpu.SemaphoreType.DMA,
        ],
        compiler_params=pltpu.CompilerParams(needs_layout_passes=False),
    )
    return f(packed, h3f)


def kernel(x, edge_index_r0, edge_index_r1, edge_index_r2,
           W0, al0, ar0, b0, W1, al1, ar1, b1, W2, al2, ar2, b2):
    Wstack = jnp.stack([W0, W1, W2])
    Astack = jnp.stack([jnp.stack([al0, ar0], axis=1),
                        jnp.stack([al1, ar1], axis=1),
                        jnp.stack([al2, ar2], axis=1)])
    h3, scores = _project(x, Wstack, Astack)

    ei = jnp.stack([edge_index_r0, edge_index_r1, edge_index_r2])
    src3 = jnp.pad(ei[:, 0, :], ((0, 0), (0, EPAD - E))).reshape(-1)
    dst3 = jnp.pad(ei[:, 1, :], ((0, 0), (0, EPAD - E)),
                   constant_values=N).reshape(-1)
    el3 = jnp.pad(scores[:, :, 0], ((0, 0), (0, NP - N))).reshape(-1)
    er3 = jnp.pad(scores[:, :, 1], ((0, 0), (0, NP - N))).reshape(-1)

    ee3, denp = _phase_a1(src3, dst3, el3, er3)
    packed = _phase_a2(src3, dst3, ee3, denp)
    u = _phase_b(packed, h3.reshape(NR * N, D))

    bsum = ((b0 + b1 + b2) / 3.0).reshape(1, D)
    return _combine(u[:N], bsum)


# speedup vs baseline: 8.5347x; 1.0047x over previous
import functools

import jax
import jax.numpy as jnp
from jax import lax
from jax.experimental import pallas as pl
from jax.experimental.pallas import tpu as pltpu
from jax.experimental.pallas import tpu_sc as plsc

N = 100000
E = 500000
D = 128
NR = 3
NBLK = 400
EPAD = 500736
NCH = EPAD // 128
NCHB = 3 * EPAD // 1024
NP = 100096
NSL = NP // 16
NC_ROWS = 11264
NCHUNK = 9
UPAD = NCHUNK * NC_ROWS
ROWS_PT = NC_ROWS // 16
KA1 = (NCH + 31) // 32
FLUSH_AT = 240

_MESH = plsc.VectorSubcoreMesh(core_axis_name="c", subcore_axis_name="s",
                               num_cores=2, num_subcores=16)
_i32 = jnp.int32


def _proj_body(x_ref, w_ref, a_ref, h_ref, s_ref):
    x = x_ref[...]
    h = jnp.dot(x, w_ref[0], preferred_element_type=jnp.float32)
    h_ref[0] = h
    s_ref[0] = jnp.dot(h, a_ref[0], preferred_element_type=jnp.float32)


def _project(x, Wstack, Astack):
    return pl.pallas_call(
        _proj_body,
        grid=(N // NBLK, NR),
        in_specs=[
            pl.BlockSpec((NBLK, D), lambda i, r: (i, 0)),
            pl.BlockSpec((1, D, D), lambda i, r: (r, 0, 0)),
            pl.BlockSpec((1, D, 2), lambda i, r: (r, 0, 0)),
        ],
        out_specs=[
            pl.BlockSpec((1, NBLK, D), lambda i, r: (r, i, 0)),
            pl.BlockSpec((1, NBLK, 2), lambda i, r: (r, i, 0)),
        ],
        out_shape=[
            jax.ShapeDtypeStruct((NR, N, D), jnp.float32),
            jax.ShapeDtypeStruct((NR, N, 2), jnp.float32),
        ],
    )(x, Wstack, Astack)


def _combine_body(u_ref, b_ref, o_ref):
    o_ref[...] = u_ref[...] * (1.0 / 3.0) + b_ref[...]


def _combine(u, bsum):
    return pl.pallas_call(
        _combine_body,
        grid=(N // NBLK,),
        in_specs=[
            pl.BlockSpec((NBLK, D), lambda i: (i, 0)),
            pl.BlockSpec((1, D), lambda i: (0, 0)),
        ],
        out_specs=pl.BlockSpec((NBLK, D), lambda i: (i, 0)),
        out_shape=jax.ShapeDtypeStruct((N, D), jnp.float32),
    )(u, bsum)


def _a1_body(src_hbm, dst_hbm, el_hbm, er_hbm, ee_hbm, denp_hbm,
             el_sh, er_sh, den_sh, buf, src_v, dst_v, eev, sem):
    c = lax.axis_index("c")
    s = lax.axis_index("s")
    w32 = c * 16 + s
    zero16 = jnp.zeros((16,), jnp.float32)

    base = s * NSL
    for r in range(NR):
        pltpu.sync_copy(el_hbm.at[pl.ds(r * NP + base, NSL)], buf)
        pltpu.sync_copy(buf, el_sh[r].at[pl.ds(base, NSL)])
        pltpu.sync_copy(er_hbm.at[pl.ds(r * NP + base, NSL)], buf)
        pltpu.sync_copy(buf, er_sh[r].at[pl.ds(base, NSL)])
    def _zero_buf(i):
        buf[pl.ds(i * 16, 16)] = zero16
    pl.loop(0, NSL // 16)(_zero_buf)
    for r in range(NR):
        pltpu.sync_copy(buf, den_sh[r].at[pl.ds(base, NSL)])
    plsc.subcore_barrier()

    def _chunk(r, k):
        j = w32 + 32 * k

        @pl.when(j < NCH)
        def _():
            eb = r * EPAD + j * 128
            pltpu.sync_copy(src_hbm.at[pl.ds(eb, 128)], src_v)
            pltpu.sync_copy(dst_hbm.at[pl.ds(eb, 128)], dst_v)
            pltpu.async_copy(el_sh[r].at[src_v], eev, sem).wait()
            pltpu.async_copy(er_sh[r].at[dst_v], buf.at[pl.ds(0, 128)], sem).wait()

            def _grp(g):
                v = eev[pl.ds(g * 16, 16)] + buf[pl.ds(g * 16, 16)]
                v = jnp.where(v > 0, v, v * jnp.float32(0.2))
                eev[pl.ds(g * 16, 16)] = jnp.exp(v)
            pl.loop(0, 8)(_grp)

            pltpu.sync_copy(eev, ee_hbm.at[pl.ds(eb, 128)])
            pltpu.sync_copy(eev, den_sh[r].at[dst_v], add=True)

    for r in range(NR):
        pl.loop(0, KA1)(functools.partial(_chunk, r))

    plsc.subcore_barrier()
    for r in range(NR):
        pltpu.sync_copy(den_sh[r].at[pl.ds(base, NSL)], buf)
        pltpu.sync_copy(buf, denp_hbm.at[pl.ds((c * NR + r) * NP + base, NSL)])


def _phase_a1(src3, dst3, el3, er3):
    f = pl.kernel(
        _a1_body,
        out_type=[
            jax.ShapeDtypeStruct((NR * EPAD,), jnp.float32),
            jax.ShapeDtypeStruct((2 * NR * NP,), jnp.float32),
        ],
        mesh=_MESH,
        scratch_types=[
            [pltpu.VMEM_SHARED((NP,), jnp.float32) for _ in range(NR)],
            [pltpu.VMEM_SHARED((NP,), jnp.float32) for _ in range(NR)],
            [pltpu.VMEM_SHARED((NP,), jnp.float32) for _ in range(NR)],
            pltpu.VMEM((NSL,), jnp.float32),
            pltpu.VMEM((128,), _i32),
            pltpu.VMEM((128,), _i32),
            pltpu.VMEM((128,), jnp.float32),
            pltpu.SemaphoreType.DMA,
        ],
        compiler_params=pltpu.CompilerParams(needs_layout_passes=False),
    )
    return f(src3, dst3, el3, er3)


def _a2_body(src_hbm, dst_hbm, ee_hbm, denp_hbm, packed_hbm,
             den_sh, v0, v1, src_v, dst_v, eev, dv, av, sem):
    c = lax.axis_index("c")
    s = lax.axis_index("s")
    w32 = c * 16 + s
    base = s * NSL

    for r in range(NR):
        pltpu.sync_copy(denp_hbm.at[pl.ds(r * NP + base, NSL)], v0)
        pltpu.sync_copy(denp_hbm.at[pl.ds((NR + r) * NP + base, NSL)], v1)

        def _sum(i):
            v0[pl.ds(i * 16, 16)] = v0[pl.ds(i * 16, 16)] + v1[pl.ds(i * 16, 16)]
        pl.loop(0, NSL // 16)(_sum)
        pltpu.sync_copy(v0, den_sh[r].at[pl.ds(base, NSL)])
    plsc.subcore_barrier()

    def _chunk(r, k):
        j = w32 + 32 * k

        @pl.when(j < NCH)
        def _():
            eb = r * EPAD + j * 128
            pb = (eb // 1024) * 3072 + (eb % 1024)
            pltpu.sync_copy(src_hbm.at[pl.ds(eb, 128)], src_v)
            pltpu.sync_copy(dst_hbm.at[pl.ds(eb, 128)], dst_v)
            pltpu.sync_copy(ee_hbm.at[pl.ds(eb, 128)], eev)
            pltpu.async_copy(den_sh[r].at[dst_v], dv, sem).wait()

            def _grp(g):
                sl = pl.ds(g * 16, 16)
                src_v[sl] = src_v[sl] + jnp.int32(r * N)
                av[sl] = plsc.bitcast(eev[sl] / dv[sl], _i32)
            pl.loop(0, 8)(_grp)
            pltpu.sync_copy(src_v, packed_hbm.at[pl.ds(pb, 128)])
            pltpu.sync_copy(dst_v, packed_hbm.at[pl.ds(pb + 1024, 128)])
            pltpu.sync_copy(av, packed_hbm.at[pl.ds(pb + 2048, 128)])

    for r in range(NR):
        pl.loop(0, KA1)(functools.partial(_chunk, r))


def _phase_a2(src3, dst3, ee3, denp):
    f = pl.kernel(
        _a2_body,
        out_type=jax.ShapeDtypeStruct((NCHB * 3072,), _i32),
        mesh=_MESH,
        scratch_types=[
            [pltpu.VMEM_SHARED((NP,), jnp.float32) for _ in range(NR)],
            pltpu.VMEM((NSL,), jnp.float32),
            pltpu.VMEM((NSL,), jnp.float32),
            pltpu.VMEM((128,), _i32),
            pltpu.VMEM((128,), _i32),
            pltpu.VMEM((128,), jnp.float32),
            pltpu.VMEM((128,), jnp.float32),
            pltpu.VMEM((128,), _i32),
            pltpu.SemaphoreType.DMA,
        ],
        compiler_params=pltpu.CompilerParams(needs_layout_passes=False),
    )
    return f(src3, dst3, ee3, denp)


def _b_body(packed_hbm, h_hbm, u_hbm,
            chunk_sh, st_src0, st_src1, st_dst0, st_dst1, st_a,
            rows0, rows1, pbuf, pbuf1, sem, sem2, semp0, semp1):
    c = lax.axis_index("c")
    s = lax.axis_index("s")
    zero16 = jnp.zeros((16,), jnp.float32)
    zero16i = jnp.zeros((16,), _i32)

    for q in range(8):
        st_src0[pl.ds(q * 16, 16)] = zero16i
        st_src1[pl.ds(q * 16, 16)] = zero16i
        st_dst0[pl.ds(q * 16, 16)] = zero16i
        st_dst1[pl.ds(q * 16, 16)] = zero16i
    for q in range(16):
        st_a[pl.ds(q * 16, 16)] = zero16

    def _flush():
        d0 = pltpu.async_copy(h_hbm.at[st_src0], rows0, sem)
        d1 = pltpu.async_copy(h_hbm.at[st_src1], rows1, sem2)
        d0.wait()
        d1.wait()

        def _scale0(i):
            av = plsc.load_gather(st_a, [zero16i + i])
            for q in range(8):
                rows0[i, pl.ds(q * 16, 16)] = rows0[i, pl.ds(q * 16, 16)] * av
        pl.loop(0, 128, unroll=4)(_scale0)
        ds0 = pltpu.async_copy(rows0, chunk_sh.at[st_dst0], sem, add=True)

        def _scale1(i):
            av = plsc.load_gather(st_a, [zero16i + (i + 128)])
            for q in range(8):
                rows1[i, pl.ds(q * 16, 16)] = rows1[i, pl.ds(q * 16, 16)] * av
        pl.loop(0, 128, unroll=4)(_scale1)
        pltpu.sync_copy(rows1, chunk_sh.at[st_dst1], add=True)
        ds0.wait()
        for q in range(16):
            st_a[pl.ds(q * 16, 16)] = zero16

    kbt = (NCHB - s + 15) // 16

    def _per_chunk(k, carry):
        chunk = 2 * k + c

        @pl.when(chunk < NCHUNK)
        def _():
            def _zrow(i):
                for q in range(8):
                    rows0[i, pl.ds(q * 16, 16)] = zero16
            pl.loop(0, 128)(_zrow)
            for i in range(5):
                pltpu.sync_copy(rows0,
                                chunk_sh.at[pl.ds(s * ROWS_PT + i * 128, 128), :])
            pltpu.sync_copy(rows0.at[pl.ds(0, 64), :],
                            chunk_sh.at[pl.ds(s * ROWS_PT + 640, 64), :])
        plsc.subcore_barrier()

        lo = chunk * NC_ROWS
        hi = jnp.minimum(lo + NC_ROWS, N)

        def _mk_grp(buf):
            def _grp(g, offv):
                vd = buf[pl.ds(1024 + g * 16, 16)]
                m = (vd >= lo) & (vd < hi)
                cs = jnp.cumsum(m.astype(_i32))
                cntv = lax.gather(
                    cs, (zero16i + 15)[:, None],
                    dimension_numbers=lax.GatherDimensionNumbers(
                        offset_dims=(), collapsed_slice_dims=(0,),
                        start_index_map=(0,)),
                    slice_sizes=(1,),
                    mode=lax.GatherScatterMode.PROMISE_IN_BOUNDS)
                pos = offv + cs - 1
                m0 = m & (pos < 128)
                m1 = m & (pos >= 128)
                vs = buf[pl.ds(g * 16, 16)]
                plsc.store_scatter(st_src0, [pos], vs, mask=m0)
                plsc.store_scatter(st_src1, [pos - 128], vs, mask=m1)
                vdl = vd - lo
                plsc.store_scatter(st_dst0, [pos], vdl, mask=m0)
                plsc.store_scatter(st_dst1, [pos - 128], vdl, mask=m1)
                plsc.store_scatter(st_a, [pos],
                                   plsc.bitcast(buf[pl.ds(2048 + g * 16, 16)],
                                                jnp.float32), mask=m)
                offv = offv + cntv
                flushp = jnp.any(offv >= FLUSH_AT)
                pl.when(flushp)(_flush)
                return jnp.where(flushp, 0, offv)
            return _grp

        def _pair(k3, off):
            t0 = (s + 32 * k3) * 3072
            d0 = pltpu.async_copy(packed_hbm.at[pl.ds(t0, 3072)], pbuf, semp0)
            d1 = pltpu.async_copy(packed_hbm.at[pl.ds(t0 + 16 * 3072, 3072)],
                                  pbuf1, semp1)
            d0.wait()
            off = lax.fori_loop(0, 64, _mk_grp(pbuf), off)
            d1.wait()
            return lax.fori_loop(0, 64, _mk_grp(pbuf1), off)

        def _single(k2, off):
            t = s + 16 * k2
            pltpu.sync_copy(packed_hbm.at[pl.ds(t * 3072, 3072)], pbuf)
            return lax.fori_loop(0, 64, _mk_grp(pbuf), off)

        @pl.when(chunk < NCHUNK)
        def _():
            o = lax.fori_loop(0, kbt // 2, _pair, jnp.zeros((16,), _i32))
            o = lax.fori_loop(2 * (kbt // 2), kbt, _single, o)
            _flush()
        plsc.subcore_barrier()

        @pl.when(chunk < NCHUNK)
        def _():
            rb = s * ROWS_PT
            for i in range(5):
                pltpu.sync_copy(chunk_sh.at[pl.ds(rb + i * 128, 128), :], rows0)
                pltpu.sync_copy(
                    rows0, u_hbm.at[pl.ds(chunk * NC_ROWS + rb + i * 128, 128), :])
            pltpu.sync_copy(chunk_sh.at[pl.ds(rb + 640, 64), :],
                            rows0.at[pl.ds(0, 64), :])
            pltpu.sync_copy(rows0.at[pl.ds(0, 64), :],
                            u_hbm.at[pl.ds(chunk * NC_ROWS + rb + 640, 64), :])
        plsc.subcore_barrier()
        return carry

    lax.fori_loop(0, (NCHUNK + 1) // 2, _per_chunk, 0)


def _phase_b(packed, h3f):
    f = pl.kernel(
        _b_body,
        out_type=jax.ShapeDtypeStruct((UPAD, D), jnp.float32),
        mesh=_MESH,
        scratch_types=[
            pltpu.VMEM_SHARED((NC_ROWS, D), jnp.float32),
            pltpu.VMEM((128,), _i32),
            pltpu.VMEM((128,), _i32),
            pltpu.VMEM((128,), _i32),
            pltpu.VMEM((128,), _i32),
            pltpu.VMEM((256,), jnp.float32),
            pltpu.VMEM((128, D), jnp.float32),
            pltpu.VMEM((128, D), jnp.float32),
            pltpu.VMEM((3072,), _i32),
            pltpu.VMEM((3072,), _i32),
            pltpu.SemaphoreType.DMA,
            pltpu.SemaphoreType.DMA,
            pltpu.SemaphoreType.DMA,
            pltpu.SemaphoreType.DMA,
        ],
        compiler_params=pltpu.CompilerParams(needs_layout_passes=False),
    )
    return f(packed, h3f)


def kernel(x, edge_index_r0, edge_index_r1, edge_index_r2,
           W0, al0, ar0, b0, W1, al1, ar1, b1, W2, al2, ar2, b2):
    Wstack = jnp.stack([W0, W1, W2])
    Astack = jnp.stack([jnp.stack([al0, ar0], axis=1),
                        jnp.stack([al1, ar1], axis=1),
                        jnp.stack([al2, ar2], axis=1)])
    h3, scores = _project(x, Wstack, Astack)

    ei = jnp.stack([edge_index_r0, edge_index_r1, edge_index_r2])
    src3 = jnp.pad(ei[:, 0, :], ((0, 0), (0, EPAD - E))).reshape(-1)
    dst3 = jnp.pad(ei[:, 1, :], ((0, 0), (0, EPAD - E)),
                   constant_values=N).reshape(-1)
    el3 = jnp.pad(scores[:, :, 0], ((0, 0), (0, NP - N))).reshape(-1)
    er3 = jnp.pad(scores[:, :, 1], ((0, 0), (0, NP - N))).reshape(-1)

    ee3, denp = _phase_a1(src3, dst3, el3, er3)
    packed = _phase_a2(src3, dst3, ee3, denp)
    u = _phase_b(packed, h3.reshape(NR * N, D))

    bsum = ((b0 + b1 + b2) / 3.0).reshape(1, D)
    return _combine(u[:N], bsum)


# speedup vs baseline: 9.5475x; 1.1187x over previous
import functools

import jax
import jax.numpy as jnp
from jax import lax
from jax.experimental import pallas as pl
from jax.experimental.pallas import tpu as pltpu
from jax.experimental.pallas import tpu_sc as plsc

N = 100000
E = 500000
D = 128
NR = 3
NBLK = 400
EPAD = 500736
NCH = EPAD // 128
NCHB = 3 * EPAD // 1024
NP = 100096
NSL = NP // 16
NC_ROWS = 11264
NCHUNK = 9
UPAD = NCHUNK * NC_ROWS
ROWS_PT = NC_ROWS // 16
KA1 = (NCH + 31) // 32
FLUSH_AT = 240

_MESH = plsc.VectorSubcoreMesh(core_axis_name="c", subcore_axis_name="s",
                               num_cores=2, num_subcores=16)
_i32 = jnp.int32


def _proj_body(x_ref, w_ref, a_ref, h_ref, s_ref):
    x = x_ref[...]
    h = jnp.dot(x, w_ref[0], preferred_element_type=jnp.float32)
    h_ref[0] = h
    s_ref[0] = jnp.dot(h, a_ref[0], preferred_element_type=jnp.float32)


def _project(x, Wstack, Astack):
    return pl.pallas_call(
        _proj_body,
        grid=(N // NBLK, NR),
        in_specs=[
            pl.BlockSpec((NBLK, D), lambda i, r: (i, 0)),
            pl.BlockSpec((1, D, D), lambda i, r: (r, 0, 0)),
            pl.BlockSpec((1, D, 2), lambda i, r: (r, 0, 0)),
        ],
        out_specs=[
            pl.BlockSpec((1, NBLK, D), lambda i, r: (r, i, 0)),
            pl.BlockSpec((1, NBLK, 2), lambda i, r: (r, i, 0)),
        ],
        out_shape=[
            jax.ShapeDtypeStruct((NR, N, D), jnp.float32),
            jax.ShapeDtypeStruct((NR, N, 2), jnp.float32),
        ],
    )(x, Wstack, Astack)


def _combine_body(u_ref, b_ref, o_ref):
    o_ref[...] = u_ref[...] * (1.0 / 3.0) + b_ref[...]


def _combine(u, bsum):
    return pl.pallas_call(
        _combine_body,
        grid=(N // NBLK,),
        in_specs=[
            pl.BlockSpec((NBLK, D), lambda i: (i, 0)),
            pl.BlockSpec((1, D), lambda i: (0, 0)),
        ],
        out_specs=pl.BlockSpec((NBLK, D), lambda i: (i, 0)),
        out_shape=jax.ShapeDtypeStruct((N, D), jnp.float32),
    )(u, bsum)


def _a1_body(src_hbm, dst_hbm, el_hbm, er_hbm, ee_hbm, denp_hbm,
             el_sh, er_sh, den_sh, buf, src_v, dst_v, eev, sem, sem2):
    c = lax.axis_index("c")
    s = lax.axis_index("s")
    w32 = c * 16 + s
    zero16 = jnp.zeros((16,), jnp.float32)

    base = s * NSL
    for r in range(NR):
        pltpu.sync_copy(el_hbm.at[pl.ds(r * NP + base, NSL)], buf)
        pltpu.sync_copy(buf, el_sh[r].at[pl.ds(base, NSL)])
        pltpu.sync_copy(er_hbm.at[pl.ds(r * NP + base, NSL)], buf)
        pltpu.sync_copy(buf, er_sh[r].at[pl.ds(base, NSL)])
    def _zero_buf(i):
        buf[pl.ds(i * 16, 16)] = zero16
    pl.loop(0, NSL // 16)(_zero_buf)
    for r in range(NR):
        pltpu.sync_copy(buf, den_sh[r].at[pl.ds(base, NSL)])
    plsc.subcore_barrier()

    def _chunk(r, k):
        j = w32 + 32 * k

        @pl.when(j < NCH)
        def _():
            eb = r * EPAD + j * 128
            d0 = pltpu.async_copy(src_hbm.at[pl.ds(eb, 128)], src_v, sem)
            d1 = pltpu.async_copy(dst_hbm.at[pl.ds(eb, 128)], dst_v, sem2)
            d0.wait()
            d2 = pltpu.async_copy(el_sh[r].at[src_v], eev, sem)
            d1.wait()
            d3 = pltpu.async_copy(er_sh[r].at[dst_v], buf.at[pl.ds(0, 128)], sem2)
            d2.wait()
            d3.wait()

            def _grp(g):
                v = eev[pl.ds(g * 16, 16)] + buf[pl.ds(g * 16, 16)]
                v = jnp.where(v > 0, v, v * jnp.float32(0.2))
                eev[pl.ds(g * 16, 16)] = jnp.exp(v)
            pl.loop(0, 8)(_grp)

            d4 = pltpu.async_copy(eev, ee_hbm.at[pl.ds(eb, 128)], sem)
            pltpu.sync_copy(eev, den_sh[r].at[dst_v], add=True)
            d4.wait()

    for r in range(NR):
        pl.loop(0, KA1)(functools.partial(_chunk, r))

    plsc.subcore_barrier()
    for r in range(NR):
        pltpu.sync_copy(den_sh[r].at[pl.ds(base, NSL)], buf)
        pltpu.sync_copy(buf, denp_hbm.at[pl.ds((c * NR + r) * NP + base, NSL)])


def _phase_a1(src3, dst3, el3, er3):
    f = pl.kernel(
        _a1_body,
        out_type=[
            jax.ShapeDtypeStruct((NR * EPAD,), jnp.float32),
            jax.ShapeDtypeStruct((2 * NR * NP,), jnp.float32),
        ],
        mesh=_MESH,
        scratch_types=[
            [pltpu.VMEM_SHARED((NP,), jnp.float32) for _ in range(NR)],
            [pltpu.VMEM_SHARED((NP,), jnp.float32) for _ in range(NR)],
            [pltpu.VMEM_SHARED((NP,), jnp.float32) for _ in range(NR)],
            pltpu.VMEM((NSL,), jnp.float32),
            pltpu.VMEM((128,), _i32),
            pltpu.VMEM((128,), _i32),
            pltpu.VMEM((128,), jnp.float32),
            pltpu.SemaphoreType.DMA,
            pltpu.SemaphoreType.DMA,
        ],
        compiler_params=pltpu.CompilerParams(needs_layout_passes=False),
    )
    return f(src3, dst3, el3, er3)


def _a2_body(src_hbm, dst_hbm, ee_hbm, denp_hbm, packed_hbm,
             den_sh, v0, v1, src_v, dst_v, eev, dv, av, sem, sem2, sem3):
    c = lax.axis_index("c")
    s = lax.axis_index("s")
    w32 = c * 16 + s
    base = s * NSL

    for r in range(NR):
        pltpu.sync_copy(denp_hbm.at[pl.ds(r * NP + base, NSL)], v0)
        pltpu.sync_copy(denp_hbm.at[pl.ds((NR + r) * NP + base, NSL)], v1)

        def _sum(i):
            v0[pl.ds(i * 16, 16)] = v0[pl.ds(i * 16, 16)] + v1[pl.ds(i * 16, 16)]
        pl.loop(0, NSL // 16)(_sum)
        pltpu.sync_copy(v0, den_sh[r].at[pl.ds(base, NSL)])
    plsc.subcore_barrier()

    def _chunk(r, k):
        j = w32 + 32 * k

        @pl.when(j < NCH)
        def _():
            eb = r * EPAD + j * 128
            pb = (eb // 1024) * 3072 + (eb % 1024)
            d0 = pltpu.async_copy(src_hbm.at[pl.ds(eb, 128)], src_v, sem)
            d1 = pltpu.async_copy(dst_hbm.at[pl.ds(eb, 128)], dst_v, sem2)
            d2 = pltpu.async_copy(ee_hbm.at[pl.ds(eb, 128)], eev, sem3)
            d1.wait()
            d3 = pltpu.async_copy(den_sh[r].at[dst_v], dv, sem2)
            d0.wait()
            d2.wait()
            d3.wait()

            def _grp(g):
                sl = pl.ds(g * 16, 16)
                src_v[sl] = src_v[sl] + jnp.int32(r * N)
                av[sl] = plsc.bitcast(eev[sl] / dv[sl], _i32)
            pl.loop(0, 8)(_grp)
            d4 = pltpu.async_copy(src_v, packed_hbm.at[pl.ds(pb, 128)], sem)
            d5 = pltpu.async_copy(dst_v, packed_hbm.at[pl.ds(pb + 1024, 128)], sem2)
            d6 = pltpu.async_copy(av, packed_hbm.at[pl.ds(pb + 2048, 128)], sem3)
            d4.wait()
            d5.wait()
            d6.wait()

    for r in range(NR):
        pl.loop(0, KA1)(functools.partial(_chunk, r))


def _phase_a2(src3, dst3, ee3, denp):
    f = pl.kernel(
        _a2_body,
        out_type=jax.ShapeDtypeStruct((NCHB * 3072,), _i32),
        mesh=_MESH,
        scratch_types=[
            [pltpu.VMEM_SHARED((NP,), jnp.float32) for _ in range(NR)],
            pltpu.VMEM((NSL,), jnp.float32),
            pltpu.VMEM((NSL,), jnp.float32),
            pltpu.VMEM((128,), _i32),
            pltpu.VMEM((128,), _i32),
            pltpu.VMEM((128,), jnp.float32),
            pltpu.VMEM((128,), jnp.float32),
            pltpu.VMEM((128,), _i32),
            pltpu.SemaphoreType.DMA,
            pltpu.SemaphoreType.DMA,
            pltpu.SemaphoreType.DMA,
        ],
        compiler_params=pltpu.CompilerParams(needs_layout_passes=False),
    )
    return f(src3, dst3, ee3, denp)


def _b_body(packed_hbm, h_hbm, u_hbm,
            chunk_sh, st_src0, st_src1, st_dst0, st_dst1, st_a,
            rows0, rows1, pbuf, pbuf1, sem, sem2, semp0, semp1):
    c = lax.axis_index("c")
    s = lax.axis_index("s")
    zero16 = jnp.zeros((16,), jnp.float32)
    zero16i = jnp.zeros((16,), _i32)

    for q in range(8):
        st_src0[pl.ds(q * 16, 16)] = zero16i
        st_src1[pl.ds(q * 16, 16)] = zero16i
        st_dst0[pl.ds(q * 16, 16)] = zero16i
        st_dst1[pl.ds(q * 16, 16)] = zero16i
    for q in range(16):
        st_a[pl.ds(q * 16, 16)] = zero16

    def _flush():
        d0 = pltpu.async_copy(h_hbm.at[st_src0], rows0, sem)
        d1 = pltpu.async_copy(h_hbm.at[st_src1], rows1, sem2)
        d0.wait()
        d1.wait()

        def _scale0(i):
            av = plsc.load_gather(st_a, [zero16i + i])
            for q in range(8):
                rows0[i, pl.ds(q * 16, 16)] = rows0[i, pl.ds(q * 16, 16)] * av
        pl.loop(0, 128, unroll=4)(_scale0)
        ds0 = pltpu.async_copy(rows0, chunk_sh.at[st_dst0], sem, add=True)

        def _scale1(i):
            av = plsc.load_gather(st_a, [zero16i + (i + 128)])
            for q in range(8):
                rows1[i, pl.ds(q * 16, 16)] = rows1[i, pl.ds(q * 16, 16)] * av
        pl.loop(0, 128, unroll=4)(_scale1)
        pltpu.sync_copy(rows1, chunk_sh.at[st_dst1], add=True)
        ds0.wait()
        for q in range(16):
            st_a[pl.ds(q * 16, 16)] = zero16

    kbt = (NCHB - s + 15) // 16

    def _per_chunk(k, carry):
        chunk = 2 * k + c

        @pl.when(chunk < NCHUNK)
        def _():
            def _zrow(i):
                for q in range(8):
                    rows0[i, pl.ds(q * 16, 16)] = zero16
            pl.loop(0, 128)(_zrow)
            for i in range(5):
                pltpu.sync_copy(rows0,
                                chunk_sh.at[pl.ds(s * ROWS_PT + i * 128, 128), :])
            pltpu.sync_copy(rows0.at[pl.ds(0, 64), :],
                            chunk_sh.at[pl.ds(s * ROWS_PT + 640, 64), :])
        plsc.subcore_barrier()

        lo = chunk * NC_ROWS
        hi = jnp.minimum(lo + NC_ROWS, N)

        def _mk_grp(buf):
            def _grp(g, offv):
                vd = buf[pl.ds(1024 + g * 16, 16)]
                m = (vd >= lo) & (vd < hi)
                cs = jnp.cumsum(m.astype(_i32))
                cntv = lax.gather(
                    cs, (zero16i + 15)[:, None],
                    dimension_numbers=lax.GatherDimensionNumbers(
                        offset_dims=(), collapsed_slice_dims=(0,),
                        start_index_map=(0,)),
                    slice_sizes=(1,),
                    mode=lax.GatherScatterMode.PROMISE_IN_BOUNDS)
                pos = offv + cs - 1
                m0 = m & (pos < 128)
                m1 = m & (pos >= 128)
                vs = buf[pl.ds(g * 16, 16)]
                plsc.store_scatter(st_src0, [pos], vs, mask=m0)
                plsc.store_scatter(st_src1, [pos - 128], vs, mask=m1)
                vdl = vd - lo
                plsc.store_scatter(st_dst0, [pos], vdl, mask=m0)
                plsc.store_scatter(st_dst1, [pos - 128], vdl, mask=m1)
                plsc.store_scatter(st_a, [pos],
                                   plsc.bitcast(buf[pl.ds(2048 + g * 16, 16)],
                                                jnp.float32), mask=m)
                offv = offv + cntv
                flushp = jnp.any(offv >= FLUSH_AT)
                pl.when(flushp)(_flush)
                return jnp.where(flushp, 0, offv)
            return _grp

        def _pair(k3, off):
            t0 = (s + 32 * k3) * 3072
            d0 = pltpu.async_copy(packed_hbm.at[pl.ds(t0, 3072)], pbuf, semp0)
            d1 = pltpu.async_copy(packed_hbm.at[pl.ds(t0 + 16 * 3072, 3072)],
                                  pbuf1, semp1)
            d0.wait()
            off = lax.fori_loop(0, 64, _mk_grp(pbuf), off)
            d1.wait()
            return lax.fori_loop(0, 64, _mk_grp(pbuf1), off)

        def _single(k2, off):
            t = s + 16 * k2
            pltpu.sync_copy(packed_hbm.at[pl.ds(t * 3072, 3072)], pbuf)
            return lax.fori_loop(0, 64, _mk_grp(pbuf), off)

        @pl.when(chunk < NCHUNK)
        def _():
            o = lax.fori_loop(0, kbt // 2, _pair, jnp.zeros((16,), _i32))
            o = lax.fori_loop(2 * (kbt // 2), kbt, _single, o)
            _flush()
        plsc.subcore_barrier()

        @pl.when(chunk < NCHUNK)
        def _():
            rb = s * ROWS_PT
            for i in range(5):
                pltpu.sync_copy(chunk_sh.at[pl.ds(rb + i * 128, 128), :], rows0)
                pltpu.sync_copy(
                    rows0, u_hbm.at[pl.ds(chunk * NC_ROWS + rb + i * 128, 128), :])
            pltpu.sync_copy(chunk_sh.at[pl.ds(rb + 640, 64), :],
                            rows0.at[pl.ds(0, 64), :])
            pltpu.sync_copy(rows0.at[pl.ds(0, 64), :],
                            u_hbm.at[pl.ds(chunk * NC_ROWS + rb + 640, 64), :])
        plsc.subcore_barrier()
        return carry

    lax.fori_loop(0, (NCHUNK + 1) // 2, _per_chunk, 0)


def _phase_b(packed, h3f):
    f = pl.kernel(
        _b_body,
        out_type=jax.ShapeDtypeStruct((UPAD, D), jnp.float32),
        mesh=_MESH,
        scratch_types=[
            pltpu.VMEM_SHARED((NC_ROWS, D), jnp.float32),
            pltpu.VMEM((128,), _i32),
            pltpu.VMEM((128,), _i32),
            pltpu.VMEM((128,), _i32),
            pltpu.VMEM((128,), _i32),
            pltpu.VMEM((256,), jnp.float32),
            pltpu.VMEM((128, D), jnp.float32),
            pltpu.VMEM((128, D), jnp.float32),
            pltpu.VMEM((3072,), _i32),
            pltpu.VMEM((3072,), _i32),
            pltpu.SemaphoreType.DMA,
            pltpu.SemaphoreType.DMA,
            pltpu.SemaphoreType.DMA,
            pltpu.SemaphoreType.DMA,
        ],
        compiler_params=pltpu.CompilerParams(needs_layout_passes=False),
    )
    return f(packed, h3f)


def kernel(x, edge_index_r0, edge_index_r1, edge_index_r2,
           W0, al0, ar0, b0, W1, al1, ar1, b1, W2, al2, ar2, b2):
    Wstack = jnp.stack([W0, W1, W2])
    Astack = jnp.stack([jnp.stack([al0, ar0], axis=1),
                        jnp.stack([al1, ar1], axis=1),
                        jnp.stack([al2, ar2], axis=1)])
    h3, scores = _project(x, Wstack, Astack)

    ei = jnp.stack([edge_index_r0, edge_index_r1, edge_index_r2])
    src3 = jnp.pad(ei[:, 0, :], ((0, 0), (0, EPAD - E))).reshape(-1)
    dst3 = jnp.pad(ei[:, 1, :], ((0, 0), (0, EPAD - E)),
                   constant_values=N).reshape(-1)
    el3 = jnp.pad(scores[:, :, 0], ((0, 0), (0, NP - N))).reshape(-1)
    er3 = jnp.pad(scores[:, :, 1], ((0, 0), (0, NP - N))).reshape(-1)

    ee3, denp = _phase_a1(src3, dst3, el3, er3)
    packed = _phase_a2(src3, dst3, ee3, denp)
    u = _phase_b(packed, h3.reshape(NR * N, D))

    bsum = ((b0 + b1 + b2) / 3.0).reshape(1, D)
    return _combine(u[:N], bsum)
